# baseline (speedup 1.0000x reference)
import functools

import jax
import jax.numpy as jnp
import numpy as np
from jax import lax
from jax.experimental import pallas as pl
from jax.experimental.pallas import tpu as pltpu
from jax.experimental.pallas import tpu_sc as plsc

_BB0 = 0.0
_BB1 = 1.0
_BASE_RES = 8
_N_LEVELS = 6
_LEVEL_SCALE = 1.32
_LOG2_T = 21
_T = 2 ** _LOG2_T
_RES = [int(np.floor(_BASE_RES * _LEVEL_SCALE ** l)) for l in range(_N_LEVELS)]
_SIDES = [r + 1 for r in _RES]
_N = 1048576
_WIDTH = 64
_N_FEAT_E = 16

_NC = 2
_NS = 16
_NW = _NC * _NS

_LEVEL_OFFS = np.cumsum([0] + [s ** 3 for s in _SIDES]).tolist()
_GTOT_RAW = _LEVEL_OFFS[-1]
_CHUNK_A = -(-_GTOT_RAW // (_NW * 16)) * 16
_GTOT = _CHUNK_A * _NW


def _build_hidx():
    hidx = np.zeros(_GTOT, dtype=np.int32)
    for l, s in enumerate(_SIDES):
        ii, jj, kk = np.meshgrid(np.arange(s, dtype=np.uint32),
                                 np.arange(s, dtype=np.uint32),
                                 np.arange(s, dtype=np.uint32), indexing="ij")
        h = (ii * np.uint32(1)) ^ (jj * np.uint32(2654435761)) ^ (kk * np.uint32(805459861))
        h &= np.uint32(_T - 1)
        hidx[_LEVEL_OFFS[l]:_LEVEL_OFFS[l + 1]] = (
            h.ravel().astype(np.int64) + l * _T).astype(np.int32)
    return hidx


_HIDX = _build_hidx()
_HIDX2 = np.empty(2 * _GTOT, dtype=np.int32)
_HIDX2[0::2] = 2 * _HIDX
_HIDX2[1::2] = 2 * _HIDX + 1

_MESH = plsc.VectorSubcoreMesh(core_axis_name="c", subcore_axis_name="s")


def _worker_id():
    return lax.axis_index("c") * _NS + lax.axis_index("s")


def _pack_bf16_pair(f0, f1):
    a0 = plsc.bitcast(f0, jnp.int32)
    a1 = plsc.bitcast(f1, jnp.int32)
    lo = lax.shift_right_logical(a0 + 0x8000, 16)
    hi = (a1 + 0x8000) & jnp.int32(-65536)
    return lo | hi


def _unpack_bf16_pair(w):
    f0 = plsc.bitcast(lax.shift_left(w, 16), jnp.float32)
    f1 = plsc.bitcast(w & jnp.int32(-65536), jnp.float32)
    return f0, f1


@functools.partial(
    pl.kernel,
    out_type=jax.ShapeDtypeStruct((_GTOT,), jnp.int32),
    mesh=_MESH,
    scratch_types=[
        pltpu.VMEM((2 * _CHUNK_A,), jnp.int32),
        pltpu.VMEM((2 * _CHUNK_A,), jnp.float32),
        pltpu.VMEM((_CHUNK_A,), jnp.int32),
        pltpu.SemaphoreType.DMA,
    ],
    compiler_params=pltpu.CompilerParams(needs_layout_passes=False),
)
def _compact(tab_hbm, hidx_hbm, out_hbm, idx_v, rows_v, out_v, sem):
    wid = _worker_id()
    base = wid * _CHUNK_A
    pltpu.sync_copy(hidx_hbm.at[pl.ds(2 * base, 2 * _CHUNK_A)], idx_v)
    pltpu.async_copy(tab_hbm.at[idx_v], rows_v, sem).wait()

    def body(i, carry):
        lanes = lax.iota(jnp.int32, 16) + i * 16
        f0 = plsc.load_gather(rows_v, [lanes * 2])
        f1 = plsc.load_gather(rows_v, [lanes * 2 + 1])
        out_v[pl.ds(i * 16, 16)] = _pack_bf16_pair(f0, f1)
        return carry

    lax.fori_loop(0, _CHUNK_A // 16, body, 0)
    pltpu.sync_copy(out_v, out_hbm.at[pl.ds(base, _CHUNK_A)])


_NPW = _N // _NW
_CB = 2048
_NCH = _NPW // _CB


@functools.partial(
    pl.kernel,
    out_type=jax.ShapeDtypeStruct((_N * _N_LEVELS,), jnp.int32),
    mesh=_MESH,
    scratch_types=[
        pltpu.VMEM((_GTOT,), jnp.int32),
        pltpu.VMEM((_CB,), jnp.float32),
        pltpu.VMEM((_CB,), jnp.float32),
        pltpu.VMEM((_CB,), jnp.float32),
        pltpu.VMEM((_CB * _N_LEVELS,), jnp.int32),
    ],
    compiler_params=pltpu.CompilerParams(needs_layout_passes=False),
)
def _encode(x_hbm, grid_hbm, pe_hbm, grid_v, xb0, xb1, xb2, peb):
    wid = _worker_id()
    base = wid * _NPW
    pltpu.sync_copy(grid_hbm, grid_v)

    def chunk_body(ci, carry):
        off = base + ci * _CB
        pltpu.sync_copy(x_hbm.at[pl.ds(off, _CB)], xb0)
        pltpu.sync_copy(x_hbm.at[pl.ds(_N + off, _CB)], xb1)
        pltpu.sync_copy(x_hbm.at[pl.ds(2 * _N + off, _CB)], xb2)

        def g_body(g, gcarry):
            p = g * 16
            x0 = xb0[pl.ds(p, 16)]
            x1 = xb1[pl.ds(p, 16)]
            x2 = xb2[pl.ds(p, 16)]
            pei = (lax.iota(jnp.int32, 16) + p) * _N_LEVELS
            for l in range(_N_LEVELS):
                res = _RES[l]
                s = _SIDES[l]
                rf = jnp.float32(res)
                s0 = x0 * rf
                s1 = x1 * rf
                s2 = x2 * rf
                i0 = jnp.minimum(s0.astype(jnp.int32), res - 1)
                i1 = jnp.minimum(s1.astype(jnp.int32), res - 1)
                i2 = jnp.minimum(s2.astype(jnp.int32), res - 1)
                f0 = s0 - i0.astype(jnp.float32)
                f1 = s1 - i1.astype(jnp.float32)
                f2 = s2 - i2.astype(jnp.float32)
                t0 = f0 * f0 * (3.0 - 2.0 * f0)
                t1 = f1 * f1 * (3.0 - 2.0 * f1)
                t2 = f2 * f2 * (3.0 - 2.0 * f2)
                flat = (i0 * s + i1) * s + i2 + _LEVEL_OFFS[l]

                def corner(d):
                    w = plsc.load_gather(grid_v, [flat + d])
                    return _unpack_bf16_pair(w)

                va000, vb000 = corner(0)
                va001, vb001 = corner(1)
                va010, vb010 = corner(s)
                va011, vb011 = corner(s + 1)
                va100, vb100 = corner(s * s)
                va101, vb101 = corner(s * s + 1)
                va110, vb110 = corner(s * s + s)
                va111, vb111 = corner(s * s + s + 1)

                def lerp(a, b, t):
                    return a + t * (b - a)

                m00 = lerp(va000, va001, t2)
                m01 = lerp(va010, va011, t2)
                m10 = lerp(va100, va101, t2)
                m11 = lerp(va110, va111, t2)
                n0 = lerp(m00, m01, t1)
                n1 = lerp(m10, m11, t1)
                acc0 = lerp(n0, n1, t0)
                q00 = lerp(vb000, vb001, t2)
                q01 = lerp(vb010, vb011, t2)
                q10 = lerp(vb100, vb101, t2)
                q11 = lerp(vb110, vb111, t2)
                r0 = lerp(q00, q01, t1)
                r1 = lerp(q10, q11, t1)
                acc1 = lerp(r0, r1, t0)

                plsc.store_scatter(peb, [pei + l], _pack_bf16_pair(acc0, acc1))
            return gcarry

        lax.fori_loop(0, _CB // 16, g_body, 0)
        pltpu.sync_copy(peb, pe_hbm.at[pl.ds(off * _N_LEVELS, _CB * _N_LEVELS)])
        return carry

    lax.fori_loop(0, _NCH, chunk_body, 0)


_BLK = 4096


def _mlp_body(pe_ref, e_ref, x_ref, w1a_ref, w1b_ref, w2_ref, w3_ref,
              b1_ref, b2_ref, b3_ref, o_ref):
    pe = pe_ref[...]
    eb = e_ref[...].astype(jnp.bfloat16)
    h = jnp.dot(pe, w1a_ref[...], preferred_element_type=jnp.float32)
    h = h + jnp.dot(eb, w1b_ref[...], preferred_element_type=jnp.float32)
    h = jnp.tanh(h + b1_ref[...])
    h2 = jnp.dot(h.astype(jnp.bfloat16), w2_ref[...], preferred_element_type=jnp.float32)
    h2 = jnp.tanh(h2 + b2_ref[...])
    o = jnp.dot(h2.astype(jnp.bfloat16), w3_ref[...], preferred_element_type=jnp.float32)
    o = o + b3_ref[...]
    o_ref[...] = o + x_ref[...]


def _mlp(pe_bf, e, xn, W1a, W1b, W2b, W3b, b1r, b2r, b3r):
    grid = (_N // _BLK,)
    return pl.pallas_call(
        _mlp_body,
        grid=grid,
        in_specs=[
            pl.BlockSpec((_BLK, 2 * _N_LEVELS), lambda i: (i, 0)),
            pl.BlockSpec((_BLK, _N_FEAT_E), lambda i: (i, 0)),
            pl.BlockSpec((_BLK, 3), lambda i: (i, 0)),
            pl.BlockSpec((2 * _N_LEVELS, _WIDTH), lambda i: (0, 0)),
            pl.BlockSpec((_N_FEAT_E, _WIDTH), lambda i: (0, 0)),
            pl.BlockSpec((_WIDTH, _WIDTH), lambda i: (0, 0)),
            pl.BlockSpec((_WIDTH, 3), lambda i: (0, 0)),
            pl.BlockSpec((1, _WIDTH), lambda i: (0, 0)),
            pl.BlockSpec((1, _WIDTH), lambda i: (0, 0)),
            pl.BlockSpec((1, 3), lambda i: (0, 0)),
        ],
        out_specs=pl.BlockSpec((_BLK, 3), lambda i: (i, 0)),
        out_shape=jax.ShapeDtypeStruct((_N, 3), jnp.float32),
    )(pe_bf, e, xn, W1a, W1b, W2b, W3b, b1r, b2r, b3r)


def kernel(x, e, tables, W1, b1, W2, b2, W3, b3):
    xn = (x - _BB0) / (_BB1 - _BB0)

    x_flat = xn.T.reshape(3 * _N)
    tab_flat = tables.reshape(_N_LEVELS * _T * 2)
    hidx = jnp.asarray(_HIDX2)

    grid_packed = _compact(tab_flat, hidx)
    pe_words = _encode(x_flat, grid_packed)

    pe_bf = lax.bitcast_convert_type(
        pe_words.reshape(_N, _N_LEVELS), jnp.bfloat16).reshape(_N, 2 * _N_LEVELS)

    W1a = W1[:2 * _N_LEVELS].astype(jnp.bfloat16)
    W1b = W1[2 * _N_LEVELS:].astype(jnp.bfloat16)
    W2b = W2.astype(jnp.bfloat16)
    W3b = W3.astype(jnp.bfloat16)
    b1r = b1.reshape(1, _WIDTH)
    b2r = b2.reshape(1, _WIDTH)
    b3r = b3.reshape(1, 3)

    out = _mlp(pe_bf, e.reshape(_N, _N_FEAT_E), xn, W1a, W1b, W2b, W3b, b1r, b2r, b3r)
    out = out * (_BB1 - _BB0) + _BB0
    return out.reshape(x.shape)


# speedup vs baseline: 1.0233x; 1.0233x over previous
import functools

import jax
import jax.numpy as jnp
import numpy as np
from jax import lax
from jax.experimental import pallas as pl
from jax.experimental.pallas import tpu as pltpu
from jax.experimental.pallas import tpu_sc as plsc

_BB0 = 0.0
_BB1 = 1.0
_BASE_RES = 8
_N_LEVELS = 6
_LEVEL_SCALE = 1.32
_LOG2_T = 21
_T = 2 ** _LOG2_T
_RES = [int(np.floor(_BASE_RES * _LEVEL_SCALE ** l)) for l in range(_N_LEVELS)]
_SIDES = [r + 1 for r in _RES]
_N = 1048576
_WIDTH = 64
_N_FEAT_E = 16

_NC = 2
_NS = 16
_NW = _NC * _NS

_LEVEL_OFFS = np.cumsum([0] + [s ** 3 for s in _SIDES]).tolist()
_GTOT_RAW = _LEVEL_OFFS[-1]
_CHUNK_A = -(-_GTOT_RAW // (_NW * 16)) * 16
_GTOT = _CHUNK_A * _NW


def _build_hidx():
    hidx = np.zeros(_GTOT, dtype=np.int32)
    for l, s in enumerate(_SIDES):
        ii, jj, kk = np.meshgrid(np.arange(s, dtype=np.uint32),
                                 np.arange(s, dtype=np.uint32),
                                 np.arange(s, dtype=np.uint32), indexing="ij")
        h = (ii * np.uint32(1)) ^ (jj * np.uint32(2654435761)) ^ (kk * np.uint32(805459861))
        h &= np.uint32(_T - 1)
        hidx[_LEVEL_OFFS[l]:_LEVEL_OFFS[l + 1]] = (
            h.ravel().astype(np.int64) + l * _T).astype(np.int32)
    return hidx


_HIDX = _build_hidx()
_HIDX2 = np.empty(2 * _GTOT, dtype=np.int32)
_HIDX2[0::2] = 2 * _HIDX
_HIDX2[1::2] = 2 * _HIDX + 1

_MESH = plsc.VectorSubcoreMesh(core_axis_name="c", subcore_axis_name="s")


def _worker_id():
    return lax.axis_index("c") * _NS + lax.axis_index("s")


def _pack_bf16_pair(f0, f1):
    a0 = plsc.bitcast(f0, jnp.int32)
    a1 = plsc.bitcast(f1, jnp.int32)
    lo = lax.shift_right_logical(a0 + 0x8000, 16)
    hi = (a1 + 0x8000) & jnp.int32(-65536)
    return lo | hi


def _unpack_bf16_pair(w):
    f0 = plsc.bitcast(lax.shift_left(w, 16), jnp.float32)
    f1 = plsc.bitcast(w & jnp.int32(-65536), jnp.float32)
    return f0, f1


@functools.partial(
    pl.kernel,
    out_type=jax.ShapeDtypeStruct((_GTOT,), jnp.int32),
    mesh=_MESH,
    scratch_types=[
        pltpu.VMEM((2 * _CHUNK_A,), jnp.int32),
        pltpu.VMEM((2 * _CHUNK_A,), jnp.float32),
        pltpu.VMEM((_CHUNK_A,), jnp.int32),
        pltpu.SemaphoreType.DMA,
    ],
    compiler_params=pltpu.CompilerParams(needs_layout_passes=False),
)
def _compact(tab_hbm, hidx_hbm, out_hbm, idx_v, rows_v, out_v, sem):
    wid = _worker_id()
    base = wid * _CHUNK_A
    pltpu.sync_copy(hidx_hbm.at[pl.ds(2 * base, 2 * _CHUNK_A)], idx_v)
    pltpu.async_copy(tab_hbm.at[idx_v], rows_v, sem).wait()

    def body(i, carry):
        lanes = lax.iota(jnp.int32, 16) + i * 16
        f0 = plsc.load_gather(rows_v, [lanes * 2])
        f1 = plsc.load_gather(rows_v, [lanes * 2 + 1])
        out_v[pl.ds(i * 16, 16)] = _pack_bf16_pair(f0, f1)
        return carry

    lax.fori_loop(0, _CHUNK_A // 16, body, 0)
    pltpu.sync_copy(out_v, out_hbm.at[pl.ds(base, _CHUNK_A)])


_NPW = _N // _NW
_CB = 2048
_NCH = _NPW // _CB


@functools.partial(
    pl.kernel,
    out_type=jax.ShapeDtypeStruct((_N * _N_LEVELS,), jnp.int32),
    mesh=_MESH,
    scratch_types=[
        pltpu.VMEM((_GTOT,), jnp.int32),
        pltpu.VMEM((3 * _CB,), jnp.float32),
        pltpu.VMEM((_CB * _N_LEVELS,), jnp.int32),
    ],
    compiler_params=pltpu.CompilerParams(needs_layout_passes=False),
)
def _encode(x_hbm, grid_hbm, pe_hbm, grid_v, xb, peb):
    wid = _worker_id()
    base = wid * _NPW
    pltpu.sync_copy(grid_hbm, grid_v)

    def chunk_body(ci, carry):
        off = base + ci * _CB
        pltpu.sync_copy(x_hbm.at[pl.ds(3 * off, 3 * _CB)], xb)

        def g_body(g, gcarry):
            p = g * 16
            lane3 = (lax.iota(jnp.int32, 16) + p) * 3
            x0 = plsc.load_gather(xb, [lane3])
            x1 = plsc.load_gather(xb, [lane3 + 1])
            x2 = plsc.load_gather(xb, [lane3 + 2])
            pei = (lax.iota(jnp.int32, 16) + p) * _N_LEVELS
            for l in range(_N_LEVELS):
                res = _RES[l]
                s = _SIDES[l]
                rf = jnp.float32(res)
                s0 = x0 * rf
                s1 = x1 * rf
                s2 = x2 * rf
                i0 = jnp.minimum(s0.astype(jnp.int32), res - 1)
                i1 = jnp.minimum(s1.astype(jnp.int32), res - 1)
                i2 = jnp.minimum(s2.astype(jnp.int32), res - 1)
                f0 = s0 - i0.astype(jnp.float32)
                f1 = s1 - i1.astype(jnp.float32)
                f2 = s2 - i2.astype(jnp.float32)
                t0 = f0 * f0 * (3.0 - 2.0 * f0)
                t1 = f1 * f1 * (3.0 - 2.0 * f1)
                t2 = f2 * f2 * (3.0 - 2.0 * f2)
                flat = (i0 * s + i1) * s + i2 + _LEVEL_OFFS[l]

                def corner(d):
                    w = plsc.load_gather(grid_v, [flat + d])
                    return _unpack_bf16_pair(w)

                va000, vb000 = corner(0)
                va001, vb001 = corner(1)
                va010, vb010 = corner(s)
                va011, vb011 = corner(s + 1)
                va100, vb100 = corner(s * s)
                va101, vb101 = corner(s * s + 1)
                va110, vb110 = corner(s * s + s)
                va111, vb111 = corner(s * s + s + 1)

                def lerp(a, b, t):
                    return a + t * (b - a)

                m00 = lerp(va000, va001, t2)
                m01 = lerp(va010, va011, t2)
                m10 = lerp(va100, va101, t2)
                m11 = lerp(va110, va111, t2)
                n0 = lerp(m00, m01, t1)
                n1 = lerp(m10, m11, t1)
                acc0 = lerp(n0, n1, t0)
                q00 = lerp(vb000, vb001, t2)
                q01 = lerp(vb010, vb011, t2)
                q10 = lerp(vb100, vb101, t2)
                q11 = lerp(vb110, vb111, t2)
                r0 = lerp(q00, q01, t1)
                r1 = lerp(q10, q11, t1)
                acc1 = lerp(r0, r1, t0)

                plsc.store_scatter(peb, [pei + l], _pack_bf16_pair(acc0, acc1))
            return gcarry

        lax.fori_loop(0, _CB // 16, g_body, 0)
        pltpu.sync_copy(peb, pe_hbm.at[pl.ds(off * _N_LEVELS, _CB * _N_LEVELS)])
        return carry

    lax.fori_loop(0, _NCH, chunk_body, 0)


_BLK = 4096


def _mlp_body(pe_ref, e_ref, x_ref, w1lo_ref, w1hi_ref, w1b_ref, w2_ref, w3_ref,
              b1_ref, b2_ref, b3_ref, o_ref):
    w = pe_ref[...]
    lo = lax.bitcast_convert_type(lax.shift_left(w, 16), jnp.float32)
    hi = lax.bitcast_convert_type(w & jnp.int32(-65536), jnp.float32)
    eb = e_ref[...].astype(jnp.bfloat16)
    h = jnp.dot(lo.astype(jnp.bfloat16), w1lo_ref[...], preferred_element_type=jnp.float32)
    h = h + jnp.dot(hi.astype(jnp.bfloat16), w1hi_ref[...], preferred_element_type=jnp.float32)
    h = h + jnp.dot(eb, w1b_ref[...], preferred_element_type=jnp.float32)
    h = jnp.tanh(h + b1_ref[...])
    h2 = jnp.dot(h.astype(jnp.bfloat16), w2_ref[...], preferred_element_type=jnp.float32)
    h2 = jnp.tanh(h2 + b2_ref[...])
    o = jnp.dot(h2.astype(jnp.bfloat16), w3_ref[...], preferred_element_type=jnp.float32)
    o = o + b3_ref[...]
    xn = (x_ref[...] - _BB0) * (1.0 / (_BB1 - _BB0))
    o_ref[...] = (o + xn) * (_BB1 - _BB0) + _BB0


def _mlp(pe_words, e, x, W1lo, W1hi, W1b, W2b, W3b, b1r, b2r, b3r):
    grid = (_N // _BLK,)
    return pl.pallas_call(
        _mlp_body,
        grid=grid,
        in_specs=[
            pl.BlockSpec((_BLK, _N_LEVELS), lambda i: (i, 0)),
            pl.BlockSpec((_BLK, _N_FEAT_E), lambda i: (i, 0)),
            pl.BlockSpec((_BLK, 3), lambda i: (i, 0)),
            pl.BlockSpec((_N_LEVELS, _WIDTH), lambda i: (0, 0)),
            pl.BlockSpec((_N_LEVELS, _WIDTH), lambda i: (0, 0)),
            pl.BlockSpec((_N_FEAT_E, _WIDTH), lambda i: (0, 0)),
            pl.BlockSpec((_WIDTH, _WIDTH), lambda i: (0, 0)),
            pl.BlockSpec((_WIDTH, 3), lambda i: (0, 0)),
            pl.BlockSpec((1, _WIDTH), lambda i: (0, 0)),
            pl.BlockSpec((1, _WIDTH), lambda i: (0, 0)),
            pl.BlockSpec((1, 3), lambda i: (0, 0)),
        ],
        out_specs=pl.BlockSpec((_BLK, 3), lambda i: (i, 0)),
        out_shape=jax.ShapeDtypeStruct((_N, 3), jnp.float32),
    )(pe_words, e, x, W1lo, W1hi, W1b, W2b, W3b, b1r, b2r, b3r)


def kernel(x, e, tables, W1, b1, W2, b2, W3, b3):
    x_flat = x.reshape(3 * _N)
    tab_flat = tables.reshape(_N_LEVELS * _T * 2)
    hidx = jnp.asarray(_HIDX2)

    grid_packed = _compact(tab_flat, hidx)
    pe_words = _encode(x_flat, grid_packed).reshape(_N, _N_LEVELS)

    W1lo = W1[0:2 * _N_LEVELS:2].astype(jnp.bfloat16)
    W1hi = W1[1:2 * _N_LEVELS:2].astype(jnp.bfloat16)
    W1b = W1[2 * _N_LEVELS:].astype(jnp.bfloat16)
    W2b = W2.astype(jnp.bfloat16)
    W3b = W3.astype(jnp.bfloat16)
    b1r = b1.reshape(1, _WIDTH)
    b2r = b2.reshape(1, _WIDTH)
    b3r = b3.reshape(1, 3)

    out = _mlp(pe_words, e.reshape(_N, _N_FEAT_E), x.reshape(_N, 3),
               W1lo, W1hi, W1b, W2b, W3b, b1r, b2r, b3r)
    return out.reshape(x.shape)


# speedup vs baseline: 14.7916x; 14.4546x over previous
import functools

import jax
import jax.numpy as jnp
import numpy as np
from jax import lax
from jax.experimental import pallas as pl
from jax.experimental.pallas import tpu as pltpu
from jax.experimental.pallas import tpu_sc as plsc

_BB0 = 0.0
_BB1 = 1.0
_BASE_RES = 8
_N_LEVELS = 6
_LEVEL_SCALE = 1.32
_LOG2_T = 21
_T = 2 ** _LOG2_T
_RES = [int(np.floor(_BASE_RES * _LEVEL_SCALE ** l)) for l in range(_N_LEVELS)]
_SIDES = [r + 1 for r in _RES]
_N = 1048576
_WIDTH = 64
_N_FEAT_E = 16

_NC = 2
_NS = 16
_NW = _NC * _NS

_LEVEL_OFFS = np.cumsum([0] + [s ** 3 for s in _SIDES]).tolist()
_GTOT_RAW = _LEVEL_OFFS[-1]
_CHUNK_A = -(-_GTOT_RAW // (_NW * 16)) * 16
_GTOT = _CHUNK_A * _NW


def _build_hidx():
    hidx2 = np.zeros(2 * _GTOT, dtype=np.int32)
    for l, s in enumerate(_SIDES):
        ii, jj, kk = np.meshgrid(np.arange(s, dtype=np.uint32),
                                 np.arange(s, dtype=np.uint32),
                                 np.arange(s, dtype=np.uint32), indexing="ij")
        h = (ii * np.uint32(1)) ^ (jj * np.uint32(2654435761)) ^ (kk * np.uint32(805459861))
        h &= np.uint32(_T - 1)
        h = h.ravel().astype(np.int64)
        hidx2[2 * _LEVEL_OFFS[l]:2 * _LEVEL_OFFS[l + 1]:2] = (2 * l * _T + h).astype(np.int32)
        hidx2[2 * _LEVEL_OFFS[l] + 1:2 * _LEVEL_OFFS[l + 1]:2] = ((2 * l + 1) * _T + h).astype(np.int32)
    return hidx2


_HIDX2 = _build_hidx()

_MESH = plsc.VectorSubcoreMesh(core_axis_name="c", subcore_axis_name="s")


def _worker_id():
    return lax.axis_index("c") * _NS + lax.axis_index("s")


def _pack_bf16_pair(f0, f1):
    a0 = plsc.bitcast(f0, jnp.int32)
    a1 = plsc.bitcast(f1, jnp.int32)
    lo = lax.shift_right_logical(a0 + 0x8000, 16)
    hi = (a1 + 0x8000) & jnp.int32(-65536)
    return lo | hi


def _unpack_bf16_pair(w):
    f0 = plsc.bitcast(lax.shift_left(w, 16), jnp.float32)
    f1 = plsc.bitcast(w, jnp.float32)
    return f0, f1


@functools.partial(
    pl.kernel,
    out_type=jax.ShapeDtypeStruct((_GTOT,), jnp.int32),
    mesh=_MESH,
    scratch_types=[
        pltpu.VMEM((2 * _CHUNK_A,), jnp.int32),
        pltpu.VMEM((2 * _CHUNK_A,), jnp.float32),
        pltpu.VMEM((_CHUNK_A,), jnp.int32),
        pltpu.SemaphoreType.DMA,
    ],
    compiler_params=pltpu.CompilerParams(needs_layout_passes=False),
)
def _compact(tab_hbm, hidx_hbm, out_hbm, idx_v, rows_v, out_v, sem):
    wid = _worker_id()
    base = wid * _CHUNK_A
    pltpu.sync_copy(hidx_hbm.at[pl.ds(2 * base, 2 * _CHUNK_A)], idx_v)
    pltpu.async_copy(tab_hbm.at[idx_v], rows_v, sem).wait()

    def body(i, carry):
        lanes = lax.iota(jnp.int32, 16) + i * 16
        f0 = plsc.load_gather(rows_v, [lanes * 2])
        f1 = plsc.load_gather(rows_v, [lanes * 2 + 1])
        out_v[pl.ds(i * 16, 16)] = _pack_bf16_pair(f0, f1)
        return carry

    lax.fori_loop(0, _CHUNK_A // 16, body, 0)
    pltpu.sync_copy(out_v, out_hbm.at[pl.ds(base, _CHUNK_A)])


_NPW = _N // _NW
_CB = 2048
_NCH = _NPW // _CB


@functools.partial(
    pl.kernel,
    out_type=jax.ShapeDtypeStruct((_N * _N_LEVELS,), jnp.int32),
    mesh=_MESH,
    scratch_types=[
        pltpu.VMEM((_GTOT,), jnp.int32),
        pltpu.VMEM((_CB,), jnp.float32),
        pltpu.VMEM((_CB,), jnp.float32),
        pltpu.VMEM((_CB,), jnp.float32),
        pltpu.VMEM((_CB * _N_LEVELS,), jnp.int32),
    ],
    compiler_params=pltpu.CompilerParams(needs_layout_passes=False),
)
def _encode(x_hbm, grid_hbm, pe_hbm, grid_v, xb0, xb1, xb2, peb):
    wid = _worker_id()
    base = wid * _NPW
    pltpu.sync_copy(grid_hbm, grid_v)

    def chunk_body(ci, carry):
        off = base + ci * _CB
        pltpu.sync_copy(x_hbm.at[pl.ds(off, _CB)], xb0)
        pltpu.sync_copy(x_hbm.at[pl.ds(_N + off, _CB)], xb1)
        pltpu.sync_copy(x_hbm.at[pl.ds(2 * _N + off, _CB)], xb2)

        def g_body(g, gcarry):
            p = g * 16
            x0 = xb0[pl.ds(p, 16)]
            x1 = xb1[pl.ds(p, 16)]
            x2 = xb2[pl.ds(p, 16)]
            for l in range(_N_LEVELS):
                res = _RES[l]
                s = _SIDES[l]
                rf = jnp.float32(res)
                s0 = x0 * rf
                s1 = x1 * rf
                s2 = x2 * rf
                i0 = s0.astype(jnp.int32)
                i1 = s1.astype(jnp.int32)
                i2 = s2.astype(jnp.int32)
                f0 = s0 - i0.astype(jnp.float32)
                f1 = s1 - i1.astype(jnp.float32)
                f2 = s2 - i2.astype(jnp.float32)
                t0 = f0 * f0 * (3.0 - 2.0 * f0)
                t1 = f1 * f1 * (3.0 - 2.0 * f1)
                t2 = f2 * f2 * (3.0 - 2.0 * f2)
                flat = (i0 * s + i1) * s + i2 + _LEVEL_OFFS[l]

                def corner(d):
                    w = plsc.load_gather(grid_v, [flat + d])
                    return _unpack_bf16_pair(w)

                va000, vb000 = corner(0)
                va001, vb001 = corner(1)
                va010, vb010 = corner(s)
                va011, vb011 = corner(s + 1)
                va100, vb100 = corner(s * s)
                va101, vb101 = corner(s * s + 1)
                va110, vb110 = corner(s * s + s)
                va111, vb111 = corner(s * s + s + 1)

                def lerp(a, b, t):
                    return a + t * (b - a)

                m00 = lerp(va000, va001, t2)
                m01 = lerp(va010, va011, t2)
                m10 = lerp(va100, va101, t2)
                m11 = lerp(va110, va111, t2)
                n0 = lerp(m00, m01, t1)
                n1 = lerp(m10, m11, t1)
                acc0 = lerp(n0, n1, t0)
                q00 = lerp(vb000, vb001, t2)
                q01 = lerp(vb010, vb011, t2)
                q10 = lerp(vb100, vb101, t2)
                q11 = lerp(vb110, vb111, t2)
                r0 = lerp(q00, q01, t1)
                r1 = lerp(q10, q11, t1)
                acc1 = lerp(r0, r1, t0)

                peb[pl.ds(l * _CB + p, 16)] = _pack_bf16_pair(acc0, acc1)
            return gcarry

        lax.fori_loop(0, _CB // 16, g_body, 0)
        for l in range(_N_LEVELS):
            pltpu.sync_copy(peb.at[pl.ds(l * _CB, _CB)],
                            pe_hbm.at[pl.ds(l * _N + off, _CB)])
        return carry

    lax.fori_loop(0, _NCH, chunk_body, 0)


_BLK = 4096


def _contract0(a, b):
    return lax.dot_general(a, b, (((0,), (0,)), ((), ())),
                           preferred_element_type=jnp.float32)


def _mlp_body(pe_ref, e_ref, x_ref, w1lo_ref, w1hi_ref, w1b_ref, w2_ref, w3_ref,
              b1_ref, b2_ref, b3_ref, o_ref):
    w = pe_ref[...]
    lo = lax.bitcast_convert_type(lax.shift_left(w, 16), jnp.float32)
    hi = lax.bitcast_convert_type(w & jnp.int32(-65536), jnp.float32)
    eb = e_ref[...].astype(jnp.bfloat16)
    h = _contract0(w1lo_ref[...], lo.astype(jnp.bfloat16))
    h = h + _contract0(w1hi_ref[...], hi.astype(jnp.bfloat16))
    h = h + _contract0(w1b_ref[...], eb)
    h = jnp.tanh(h + b1_ref[...])
    h2 = _contract0(w2_ref[...], h.astype(jnp.bfloat16))
    h2 = jnp.tanh(h2 + b2_ref[...])
    o = _contract0(w3_ref[...], h2.astype(jnp.bfloat16))
    o = o + b3_ref[...]
    xn = (x_ref[...] - _BB0) * (1.0 / (_BB1 - _BB0))
    o_ref[...] = (o + xn) * (_BB1 - _BB0) + _BB0


def _mlp(pe_t, e_t, x_t, W1lo, W1hi, W1b, W2b, W3b, b1r, b2r, b3r):
    grid = (_N // _BLK,)
    return pl.pallas_call(
        _mlp_body,
        grid=grid,
        in_specs=[
            pl.BlockSpec((_N_LEVELS, _BLK), lambda i: (0, i)),
            pl.BlockSpec((_N_FEAT_E, _BLK), lambda i: (0, i)),
            pl.BlockSpec((3, _BLK), lambda i: (0, i)),
            pl.BlockSpec((_N_LEVELS, _WIDTH), lambda i: (0, 0)),
            pl.BlockSpec((_N_LEVELS, _WIDTH), lambda i: (0, 0)),
            pl.BlockSpec((_N_FEAT_E, _WIDTH), lambda i: (0, 0)),
            pl.BlockSpec((_WIDTH, _WIDTH), lambda i: (0, 0)),
            pl.BlockSpec((_WIDTH, 3), lambda i: (0, 0)),
            pl.BlockSpec((_WIDTH, 1), lambda i: (0, 0)),
            pl.BlockSpec((_WIDTH, 1), lambda i: (0, 0)),
            pl.BlockSpec((3, 1), lambda i: (0, 0)),
        ],
        out_specs=pl.BlockSpec((3, _BLK), lambda i: (0, i)),
        out_shape=jax.ShapeDtypeStruct((3, _N), jnp.float32),
    )(pe_t, e_t, x_t, W1lo, W1hi, W1b, W2b, W3b, b1r, b2r, b3r)


def kernel(x, e, tables, W1, b1, W2, b2, W3, b3):
    x_t = x.T
    x_flat = x_t.reshape(3 * _N)
    e_t = e.T
    tab_flat = jnp.transpose(tables, (0, 2, 1)).reshape(_N_LEVELS * 2 * _T)
    hidx = jnp.asarray(_HIDX2)

    grid_packed = _compact(tab_flat, hidx)
    pe_t = _encode(x_flat, grid_packed).reshape(_N_LEVELS, _N)

    W1lo = W1[0:2 * _N_LEVELS:2].astype(jnp.bfloat16)
    W1hi = W1[1:2 * _N_LEVELS:2].astype(jnp.bfloat16)
    W1b = W1[2 * _N_LEVELS:].astype(jnp.bfloat16)
    W2b = W2.astype(jnp.bfloat16)
    W3b = W3.astype(jnp.bfloat16)
    b1r = b1.reshape(_WIDTH, 1)
    b2r = b2.reshape(_WIDTH, 1)
    b3r = b3.reshape(3, 1)

    out_t = _mlp(pe_t, e_t, x_t, W1lo, W1hi, W1b, W2b, W3b, b1r, b2r, b3r)
    return out_t.T.reshape(x.shape)


# speedup vs baseline: 24.9411x; 1.6862x over previous
import functools

import jax
import jax.numpy as jnp
import numpy as np
from jax import lax
from jax.experimental import pallas as pl
from jax.experimental.pallas import tpu as pltpu
from jax.experimental.pallas import tpu_sc as plsc

_BB0 = 0.0
_BB1 = 1.0
_BASE_RES = 8
_N_LEVELS = 6
_LEVEL_SCALE = 1.32
_LOG2_T = 21
_T = 2 ** _LOG2_T
_RES = [int(np.floor(_BASE_RES * _LEVEL_SCALE ** l)) for l in range(_N_LEVELS)]
_SIDES = [r + 1 for r in _RES]
_N = 1048576
_WIDTH = 64
_N_FEAT_E = 16

_NC = 2
_NS = 16
_NW = _NC * _NS

_LEVEL_OFFS = np.cumsum([0] + [s ** 3 for s in _SIDES]).tolist()
_GTOT_RAW = _LEVEL_OFFS[-1]
_CHUNK_A = -(-_GTOT_RAW // (_NW * 16)) * 16
_GTOT = _CHUNK_A * _NW


def _build_hidx():
    hidx2 = np.zeros(2 * _GTOT, dtype=np.int32)
    for l, s in enumerate(_SIDES):
        ii, jj, kk = np.meshgrid(np.arange(s, dtype=np.uint32),
                                 np.arange(s, dtype=np.uint32),
                                 np.arange(s, dtype=np.uint32), indexing="ij")
        h = (ii * np.uint32(1)) ^ (jj * np.uint32(2654435761)) ^ (kk * np.uint32(805459861))
        h &= np.uint32(_T - 1)
        h = h.ravel().astype(np.int64)
        hidx2[2 * _LEVEL_OFFS[l]:2 * _LEVEL_OFFS[l + 1]:2] = (2 * l * _T + h).astype(np.int32)
        hidx2[2 * _LEVEL_OFFS[l] + 1:2 * _LEVEL_OFFS[l + 1]:2] = ((2 * l + 1) * _T + h).astype(np.int32)
    return hidx2


_HIDX2 = _build_hidx()

_MESH = plsc.VectorSubcoreMesh(core_axis_name="c", subcore_axis_name="s")


def _worker_id():
    return lax.axis_index("c") * _NS + lax.axis_index("s")


def _pack_bf16_pair(f0, f1):
    a0 = plsc.bitcast(f0, jnp.int32)
    a1 = plsc.bitcast(f1, jnp.int32)
    lo = lax.shift_right_logical(a0 + 0x8000, 16)
    hi = (a1 + 0x8000) & jnp.int32(-65536)
    return lo | hi


def _unpack_bf16_pair(w):
    f0 = plsc.bitcast(lax.shift_left(w, 16), jnp.float32)
    f1 = plsc.bitcast(w, jnp.float32)
    return f0, f1


@functools.partial(
    pl.kernel,
    out_type=jax.ShapeDtypeStruct((_GTOT,), jnp.int32),
    mesh=_MESH,
    scratch_types=[
        pltpu.VMEM((2 * _CHUNK_A,), jnp.int32),
        pltpu.VMEM((2 * _CHUNK_A,), jnp.float32),
        pltpu.VMEM((_CHUNK_A,), jnp.int32),
        pltpu.SemaphoreType.DMA,
    ],
    compiler_params=pltpu.CompilerParams(needs_layout_passes=False),
)
def _compact(tab_hbm, hidx_hbm, out_hbm, idx_v, rows_v, out_v, sem):
    wid = _worker_id()
    base = wid * _CHUNK_A
    pltpu.sync_copy(hidx_hbm.at[pl.ds(2 * base, 2 * _CHUNK_A)], idx_v)
    pltpu.async_copy(tab_hbm.at[idx_v], rows_v, sem).wait()

    def body(i, carry):
        lanes = lax.iota(jnp.int32, 16) + i * 16
        f0 = plsc.load_gather(rows_v, [lanes * 2])
        f1 = plsc.load_gather(rows_v, [lanes * 2 + 1])
        out_v[pl.ds(i * 16, 16)] = _pack_bf16_pair(f0, f1)
        return carry

    lax.fori_loop(0, _CHUNK_A // 16, body, 0)
    pltpu.sync_copy(out_v, out_hbm.at[pl.ds(base, _CHUNK_A)])


_NPW = _N // _NW
_CB = 2048
_NCH = _NPW // _CB


@functools.partial(
    pl.kernel,
    out_type=jax.ShapeDtypeStruct((_N_LEVELS, _N), jnp.int32),
    mesh=_MESH,
    scratch_types=[
        pltpu.VMEM((_GTOT,), jnp.int32),
        pltpu.VMEM((_CB,), jnp.float32),
        pltpu.VMEM((_CB,), jnp.float32),
        pltpu.VMEM((_CB,), jnp.float32),
        pltpu.VMEM((_CB * _N_LEVELS,), jnp.int32),
    ],
    compiler_params=pltpu.CompilerParams(needs_layout_passes=False),
)
def _encode(x_hbm, grid_hbm, pe_hbm, grid_v, xb0, xb1, xb2, peb):
    wid = _worker_id()
    base = wid * _NPW
    pltpu.sync_copy(grid_hbm, grid_v)

    def chunk_body(ci, carry):
        off = base + ci * _CB
        pltpu.sync_copy(x_hbm.at[pl.ds(off, _CB)], xb0)
        pltpu.sync_copy(x_hbm.at[pl.ds(_N + off, _CB)], xb1)
        pltpu.sync_copy(x_hbm.at[pl.ds(2 * _N + off, _CB)], xb2)

        def g_body(g, gcarry):
            p = g * 16
            x0 = xb0[pl.ds(p, 16)]
            x1 = xb1[pl.ds(p, 16)]
            x2 = xb2[pl.ds(p, 16)]
            for l in range(_N_LEVELS):
                res = _RES[l]
                s = _SIDES[l]
                rf = jnp.float32(res)
                s0 = x0 * rf
                s1 = x1 * rf
                s2 = x2 * rf
                i0 = s0.astype(jnp.int32)
                i1 = s1.astype(jnp.int32)
                i2 = s2.astype(jnp.int32)
                f0 = s0 - i0.astype(jnp.float32)
                f1 = s1 - i1.astype(jnp.float32)
                f2 = s2 - i2.astype(jnp.float32)
                t0 = f0 * f0 * (3.0 - 2.0 * f0)
                t1 = f1 * f1 * (3.0 - 2.0 * f1)
                t2 = f2 * f2 * (3.0 - 2.0 * f2)
                flat = (i0 * s + i1) * s + i2 + _LEVEL_OFFS[l]

                def corner(d):
                    w = plsc.load_gather(grid_v, [flat + d])
                    return _unpack_bf16_pair(w)

                va000, vb000 = corner(0)
                va001, vb001 = corner(1)
                va010, vb010 = corner(s)
                va011, vb011 = corner(s + 1)
                va100, vb100 = corner(s * s)
                va101, vb101 = corner(s * s + 1)
                va110, vb110 = corner(s * s + s)
                va111, vb111 = corner(s * s + s + 1)

                def lerp(a, b, t):
                    return a + t * (b - a)

                m00 = lerp(va000, va001, t2)
                m01 = lerp(va010, va011, t2)
                m10 = lerp(va100, va101, t2)
                m11 = lerp(va110, va111, t2)
                n0 = lerp(m00, m01, t1)
                n1 = lerp(m10, m11, t1)
                acc0 = lerp(n0, n1, t0)
                q00 = lerp(vb000, vb001, t2)
                q01 = lerp(vb010, vb011, t2)
                q10 = lerp(vb100, vb101, t2)
                q11 = lerp(vb110, vb111, t2)
                r0 = lerp(q00, q01, t1)
                r1 = lerp(q10, q11, t1)
                acc1 = lerp(r0, r1, t0)

                peb[pl.ds(l * _CB + p, 16)] = _pack_bf16_pair(acc0, acc1)
            return gcarry

        lax.fori_loop(0, _CB // 16, g_body, 0)
        for l in range(_N_LEVELS):
            pltpu.sync_copy(peb.at[pl.ds(l * _CB, _CB)],
                            pe_hbm.at[l, pl.ds(off, _CB)])
        return carry

    lax.fori_loop(0, _NCH, chunk_body, 0)


_BLK = 4096


def _contract0(a, b):
    return lax.dot_general(a, b, (((0,), (0,)), ((), ())),
                           preferred_element_type=jnp.float32)


def _mlp_body(pe_ref, e_ref, x_ref, w1lo_ref, w1hi_ref, w1b_ref, w2_ref, w3_ref,
              b1_ref, b2_ref, b3_ref, o_ref):
    w = pe_ref[...]
    lo = lax.bitcast_convert_type(lax.shift_left(w, 16), jnp.float32)
    hi = lax.bitcast_convert_type(w & jnp.int32(-65536), jnp.float32)
    eb = e_ref[...].astype(jnp.bfloat16)
    h = _contract0(w1lo_ref[...], lo.astype(jnp.bfloat16))
    h = h + _contract0(w1hi_ref[...], hi.astype(jnp.bfloat16))
    h = h + _contract0(w1b_ref[...], eb)
    h = jnp.tanh(h + b1_ref[...])
    h2 = _contract0(w2_ref[...], h.astype(jnp.bfloat16))
    h2 = jnp.tanh(h2 + b2_ref[...])
    o = _contract0(w3_ref[...], h2.astype(jnp.bfloat16))
    o = o + b3_ref[...]
    xn = (x_ref[...] - _BB0) * (1.0 / (_BB1 - _BB0))
    o_ref[...] = (o + xn) * (_BB1 - _BB0) + _BB0


def _mlp(pe_t, e_t, x_t, W1lo, W1hi, W1b, W2b, W3b, b1r, b2r, b3r):
    grid = (_N // _BLK,)
    return pl.pallas_call(
        _mlp_body,
        grid=grid,
        in_specs=[
            pl.BlockSpec((_N_LEVELS, _BLK), lambda i: (0, i)),
            pl.BlockSpec((_N_FEAT_E, _BLK), lambda i: (0, i)),
            pl.BlockSpec((3, _BLK), lambda i: (0, i)),
            pl.BlockSpec((_N_LEVELS, _WIDTH), lambda i: (0, 0)),
            pl.BlockSpec((_N_LEVELS, _WIDTH), lambda i: (0, 0)),
            pl.BlockSpec((_N_FEAT_E, _WIDTH), lambda i: (0, 0)),
            pl.BlockSpec((_WIDTH, _WIDTH), lambda i: (0, 0)),
            pl.BlockSpec((_WIDTH, 3), lambda i: (0, 0)),
            pl.BlockSpec((_WIDTH, 1), lambda i: (0, 0)),
            pl.BlockSpec((_WIDTH, 1), lambda i: (0, 0)),
            pl.BlockSpec((3, 1), lambda i: (0, 0)),
        ],
        out_specs=pl.BlockSpec((3, _BLK), lambda i: (0, i)),
        out_shape=jax.ShapeDtypeStruct((3, _N), jnp.float32),
    )(pe_t, e_t, x_t, W1lo, W1hi, W1b, W2b, W3b, b1r, b2r, b3r)


def kernel(x, e, tables, W1, b1, W2, b2, W3, b3):
    x_t = x.T
    x_flat = x_t.reshape(3 * _N)
    e_t = e.T
    tab_flat = jnp.transpose(tables, (0, 2, 1)).reshape(_N_LEVELS * 2 * _T)
    hidx = jnp.asarray(_HIDX2)

    grid_packed = _compact(tab_flat, hidx)
    pe_t = _encode(x_flat, grid_packed)

    W1lo = W1[0:2 * _N_LEVELS:2].astype(jnp.bfloat16)
    W1hi = W1[1:2 * _N_LEVELS:2].astype(jnp.bfloat16)
    W1b = W1[2 * _N_LEVELS:].astype(jnp.bfloat16)
    W2b = W2.astype(jnp.bfloat16)
    W3b = W3.astype(jnp.bfloat16)
    b1r = b1.reshape(_WIDTH, 1)
    b2r = b2.reshape(_WIDTH, 1)
    b3r = b3.reshape(3, 1)

    out_t = _mlp(pe_t, e_t, x_t, W1lo, W1hi, W1b, W2b, W3b, b1r, b2r, b3r)
    return out_t.T.reshape(x.shape)


# speedup vs baseline: 26.6441x; 1.0683x over previous
import functools

import jax
import jax.numpy as jnp
import numpy as np
from jax import lax
from jax.experimental import pallas as pl
from jax.experimental.pallas import tpu as pltpu
from jax.experimental.pallas import tpu_sc as plsc

_BB0 = 0.0
_BB1 = 1.0
_BASE_RES = 8
_N_LEVELS = 6
_LEVEL_SCALE = 1.32
_LOG2_T = 21
_T = 2 ** _LOG2_T
_RES = [int(np.floor(_BASE_RES * _LEVEL_SCALE ** l)) for l in range(_N_LEVELS)]
_SIDES = [r + 1 for r in _RES]
_N = 1048576
_WIDTH = 64
_N_FEAT_E = 16

_NC = 2
_NS = 16
_NW = _NC * _NS

_LEVEL_OFFS = np.cumsum([0] + [s ** 3 for s in _SIDES]).tolist()
_GTOT_RAW = _LEVEL_OFFS[-1]
_CHUNK_A = -(-_GTOT_RAW // (_NW * 16)) * 16
_GTOT = _CHUNK_A * _NW


def _build_hidx():
    hidx2 = np.zeros(2 * _GTOT, dtype=np.int32)
    for l, s in enumerate(_SIDES):
        ii, jj, kk = np.meshgrid(np.arange(s, dtype=np.uint32),
                                 np.arange(s, dtype=np.uint32),
                                 np.arange(s, dtype=np.uint32), indexing="ij")
        h = (ii * np.uint32(1)) ^ (jj * np.uint32(2654435761)) ^ (kk * np.uint32(805459861))
        h &= np.uint32(_T - 1)
        h = h.ravel().astype(np.int64)
        hidx2[2 * _LEVEL_OFFS[l]:2 * _LEVEL_OFFS[l + 1]:2] = (2 * l * _T + h).astype(np.int32)
        hidx2[2 * _LEVEL_OFFS[l] + 1:2 * _LEVEL_OFFS[l + 1]:2] = ((2 * l + 1) * _T + h).astype(np.int32)
    return hidx2


_HIDX2 = _build_hidx()

_MESH = plsc.VectorSubcoreMesh(core_axis_name="c", subcore_axis_name="s")


def _worker_id():
    return lax.axis_index("c") * _NS + lax.axis_index("s")


def _pack_bf16_pair(f0, f1):
    a0 = plsc.bitcast(f0, jnp.int32)
    a1 = plsc.bitcast(f1, jnp.int32)
    lo = lax.shift_right_logical(a0 + 0x8000, 16)
    hi = (a1 + 0x8000) & jnp.int32(-65536)
    return lo | hi


def _unpack_bf16_pair(w):
    f0 = plsc.bitcast(lax.shift_left(w, 16), jnp.float32)
    f1 = plsc.bitcast(w, jnp.float32)
    return f0, f1


@functools.partial(
    pl.kernel,
    out_type=jax.ShapeDtypeStruct((_GTOT,), jnp.int32),
    mesh=_MESH,
    scratch_types=[
        pltpu.VMEM((2 * _CHUNK_A,), jnp.int32),
        pltpu.VMEM((2 * _CHUNK_A,), jnp.float32),
        pltpu.VMEM((_CHUNK_A,), jnp.int32),
        pltpu.SemaphoreType.DMA,
    ],
    compiler_params=pltpu.CompilerParams(needs_layout_passes=False),
)
def _compact(tab_hbm, hidx_hbm, out_hbm, idx_v, rows_v, out_v, sem):
    wid = _worker_id()
    base = wid * _CHUNK_A
    pltpu.sync_copy(hidx_hbm.at[pl.ds(2 * base, 2 * _CHUNK_A)], idx_v)
    pltpu.async_copy(tab_hbm.at[idx_v], rows_v, sem).wait()

    def body(i, carry):
        lanes = lax.iota(jnp.int32, 16) + i * 16
        f0 = plsc.load_gather(rows_v, [lanes * 2])
        f1 = plsc.load_gather(rows_v, [lanes * 2 + 1])
        out_v[pl.ds(i * 16, 16)] = _pack_bf16_pair(f0, f1)
        return carry

    lax.fori_loop(0, _CHUNK_A // 16, body, 0)
    pltpu.sync_copy(out_v, out_hbm.at[pl.ds(base, _CHUNK_A)])


_NPW = _N // _NW
_CB = 2048
_NCH = _NPW // _CB


@functools.partial(
    pl.kernel,
    out_type=jax.ShapeDtypeStruct((_N_LEVELS, _N), jnp.int32),
    mesh=_MESH,
    scratch_types=[
        pltpu.VMEM((_GTOT,), jnp.int32),
        pltpu.VMEM((_CB,), jnp.float32),
        pltpu.VMEM((_CB,), jnp.float32),
        pltpu.VMEM((_CB,), jnp.float32),
        pltpu.VMEM((_CB * _N_LEVELS,), jnp.int32),
    ],
    compiler_params=pltpu.CompilerParams(needs_layout_passes=False),
)
def _encode(x_hbm, grid_hbm, pe_hbm, grid_v, xb0, xb1, xb2, peb):
    wid = _worker_id()
    base = wid * _NPW
    pltpu.sync_copy(grid_hbm, grid_v)

    def chunk_body(ci, carry):
        off = base + ci * _CB
        pltpu.sync_copy(x_hbm.at[pl.ds(off, _CB)], xb0)
        pltpu.sync_copy(x_hbm.at[pl.ds(_N + off, _CB)], xb1)
        pltpu.sync_copy(x_hbm.at[pl.ds(2 * _N + off, _CB)], xb2)

        def g_body(g, gcarry):
            p = g * 16
            x0 = xb0[pl.ds(p, 16)]
            x1 = xb1[pl.ds(p, 16)]
            x2 = xb2[pl.ds(p, 16)]
            for l in range(_N_LEVELS):
                res = _RES[l]
                s = _SIDES[l]
                rf = jnp.float32(res)
                s0 = x0 * rf
                s1 = x1 * rf
                s2 = x2 * rf
                i0 = s0.astype(jnp.int32)
                i1 = s1.astype(jnp.int32)
                i2 = s2.astype(jnp.int32)
                f0 = s0 - i0.astype(jnp.float32)
                f1 = s1 - i1.astype(jnp.float32)
                f2 = s2 - i2.astype(jnp.float32)
                t0 = f0 * f0 * (3.0 - 2.0 * f0)
                t1 = f1 * f1 * (3.0 - 2.0 * f1)
                t2 = f2 * f2 * (3.0 - 2.0 * f2)
                flat = (i0 * s + i1) * s + i2 + _LEVEL_OFFS[l]

                tw0 = plsc.pack(t0, t0, format=plsc.PackFormat.INTERLEAVED)
                tw1 = plsc.pack(t1, t1, format=plsc.PackFormat.INTERLEAVED)
                tw2 = plsc.pack(t2, t2, format=plsc.PackFormat.INTERLEAVED)

                def corner(d):
                    w = plsc.load_gather(grid_v, [flat + d])
                    return plsc.bitcast(w, jnp.bfloat16)

                c000 = corner(0)
                c001 = corner(1)
                c010 = corner(s)
                c011 = corner(s + 1)
                c100 = corner(s * s)
                c101 = corner(s * s + 1)
                c110 = corner(s * s + s)
                c111 = corner(s * s + s + 1)

                def lerp(a, b, t):
                    return a + t * (b - a)

                m00 = lerp(c000, c001, tw2)
                m01 = lerp(c010, c011, tw2)
                m10 = lerp(c100, c101, tw2)
                m11 = lerp(c110, c111, tw2)
                n0 = lerp(m00, m01, tw1)
                n1 = lerp(m10, m11, tw1)
                acc = lerp(n0, n1, tw0)

                peb[pl.ds(l * _CB + p, 16)] = plsc.bitcast(acc, jnp.int32)
            return gcarry

        lax.fori_loop(0, _CB // 16, g_body, 0)
        for l in range(_N_LEVELS):
            pltpu.sync_copy(peb.at[pl.ds(l * _CB, _CB)],
                            pe_hbm.at[l, pl.ds(off, _CB)])
        return carry

    lax.fori_loop(0, _NCH, chunk_body, 0)


_BLK = 4096


def _contract0(a, b):
    return lax.dot_general(a, b, (((0,), (0,)), ((), ())),
                           preferred_element_type=jnp.float32)


def _mlp_body(pe_ref, e_ref, x_ref, w1lo_ref, w1hi_ref, w1b_ref, w2_ref, w3_ref,
              b1_ref, b2_ref, b3_ref, o_ref):
    w = pe_ref[...]
    lo = lax.bitcast_convert_type(lax.shift_left(w, 16), jnp.float32)
    hi = lax.bitcast_convert_type(w & jnp.int32(-65536), jnp.float32)
    eb = e_ref[...].astype(jnp.bfloat16)
    h = _contract0(w1lo_ref[...], lo.astype(jnp.bfloat16))
    h = h + _contract0(w1hi_ref[...], hi.astype(jnp.bfloat16))
    h = h + _contract0(w1b_ref[...], eb)
    h = jnp.tanh(h + b1_ref[...])
    h2 = _contract0(w2_ref[...], h.astype(jnp.bfloat16))
    h2 = jnp.tanh(h2 + b2_ref[...])
    o = _contract0(w3_ref[...], h2.astype(jnp.bfloat16))
    o = o + b3_ref[...]
    xn = (x_ref[...] - _BB0) * (1.0 / (_BB1 - _BB0))
    o_ref[...] = (o + xn) * (_BB1 - _BB0) + _BB0


def _mlp(pe_t, e_t, x_t, W1lo, W1hi, W1b, W2b, W3b, b1r, b2r, b3r):
    grid = (_N // _BLK,)
    return pl.pallas_call(
        _mlp_body,
        grid=grid,
        in_specs=[
            pl.BlockSpec((_N_LEVELS, _BLK), lambda i: (0, i)),
            pl.BlockSpec((_N_FEAT_E, _BLK), lambda i: (0, i)),
            pl.BlockSpec((3, _BLK), lambda i: (0, i)),
            pl.BlockSpec((_N_LEVELS, _WIDTH), lambda i: (0, 0)),
            pl.BlockSpec((_N_LEVELS, _WIDTH), lambda i: (0, 0)),
            pl.BlockSpec((_N_FEAT_E, _WIDTH), lambda i: (0, 0)),
            pl.BlockSpec((_WIDTH, _WIDTH), lambda i: (0, 0)),
            pl.BlockSpec((_WIDTH, 3), lambda i: (0, 0)),
            pl.BlockSpec((_WIDTH, 1), lambda i: (0, 0)),
            pl.BlockSpec((_WIDTH, 1), lambda i: (0, 0)),
            pl.BlockSpec((3, 1), lambda i: (0, 0)),
        ],
        out_specs=pl.BlockSpec((3, _BLK), lambda i: (0, i)),
        out_shape=jax.ShapeDtypeStruct((3, _N), jnp.float32),
    )(pe_t, e_t, x_t, W1lo, W1hi, W1b, W2b, W3b, b1r, b2r, b3r)


def kernel(x, e, tables, W1, b1, W2, b2, W3, b3):
    x_t = x.T
    x_flat = x_t.reshape(3 * _N)
    e_t = e.T
    tab_flat = jnp.transpose(tables, (0, 2, 1)).reshape(_N_LEVELS * 2 * _T)
    hidx = jnp.asarray(_HIDX2)

    grid_packed = _compact(tab_flat, hidx)
    pe_t = _encode(x_flat, grid_packed)

    W1lo = W1[0:2 * _N_LEVELS:2].astype(jnp.bfloat16)
    W1hi = W1[1:2 * _N_LEVELS:2].astype(jnp.bfloat16)
    W1b = W1[2 * _N_LEVELS:].astype(jnp.bfloat16)
    W2b = W2.astype(jnp.bfloat16)
    W3b = W3.astype(jnp.bfloat16)
    b1r = b1.reshape(_WIDTH, 1)
    b2r = b2.reshape(_WIDTH, 1)
    b3r = b3.reshape(3, 1)

    out_t = _mlp(pe_t, e_t, x_t, W1lo, W1hi, W1b, W2b, W3b, b1r, b2r, b3r)
    return out_t.T.reshape(x.shape)


# speedup vs baseline: 30.5722x; 1.1474x over previous
import functools

import jax
import jax.numpy as jnp
import numpy as np
from jax import lax
from jax.experimental import pallas as pl
from jax.experimental.pallas import tpu as pltpu
from jax.experimental.pallas import tpu_sc as plsc

_BB0 = 0.0
_BB1 = 1.0
_BASE_RES = 8
_N_LEVELS = 6
_LEVEL_SCALE = 1.32
_LOG2_T = 21
_T = 2 ** _LOG2_T
_RES = [int(np.floor(_BASE_RES * _LEVEL_SCALE ** l)) for l in range(_N_LEVELS)]
_SIDES = [r + 1 for r in _RES]
_N = 1048576
_WIDTH = 64
_N_FEAT_E = 16

_NC = 2
_NS = 16
_NW = _NC * _NS

_LEVEL_OFFS = np.cumsum([0] + [s ** 3 for s in _SIDES]).tolist()
_GTOT_RAW = _LEVEL_OFFS[-1]
_CHUNK_A = -(-_GTOT_RAW // (_NW * 16)) * 16
_GTOT = _CHUNK_A * _NW


def _build_hidx():
    hidx2 = np.zeros(2 * _GTOT, dtype=np.int32)
    for l, s in enumerate(_SIDES):
        ii, jj, kk = np.meshgrid(np.arange(s, dtype=np.uint32),
                                 np.arange(s, dtype=np.uint32),
                                 np.arange(s, dtype=np.uint32), indexing="ij")
        h = (ii * np.uint32(1)) ^ (jj * np.uint32(2654435761)) ^ (kk * np.uint32(805459861))
        h &= np.uint32(_T - 1)
        h = h.ravel().astype(np.int64)
        hidx2[2 * _LEVEL_OFFS[l]:2 * _LEVEL_OFFS[l + 1]:2] = (2 * l * _T + h).astype(np.int32)
        hidx2[2 * _LEVEL_OFFS[l] + 1:2 * _LEVEL_OFFS[l + 1]:2] = ((2 * l + 1) * _T + h).astype(np.int32)
    return hidx2


_HIDX2 = _build_hidx()

_MESH = plsc.VectorSubcoreMesh(core_axis_name="c", subcore_axis_name="s")


def _worker_id():
    return lax.axis_index("c") * _NS + lax.axis_index("s")


def _pack_bf16_pair(f0, f1):
    a0 = plsc.bitcast(f0, jnp.int32)
    a1 = plsc.bitcast(f1, jnp.int32)
    lo = lax.shift_right_logical(a0 + 0x8000, 16)
    hi = (a1 + 0x8000) & jnp.int32(-65536)
    return lo | hi


def _unpack_bf16_pair(w):
    f0 = plsc.bitcast(lax.shift_left(w, 16), jnp.float32)
    f1 = plsc.bitcast(w, jnp.float32)
    return f0, f1


@functools.partial(
    pl.kernel,
    out_type=jax.ShapeDtypeStruct((_GTOT,), jnp.int32),
    mesh=_MESH,
    scratch_types=[
        pltpu.VMEM((2 * _CHUNK_A,), jnp.int32),
        pltpu.VMEM((2 * _CHUNK_A,), jnp.float32),
        pltpu.VMEM((_CHUNK_A,), jnp.int32),
        pltpu.SemaphoreType.DMA,
    ],
    compiler_params=pltpu.CompilerParams(needs_layout_passes=False),
)
def _compact(tab_hbm, hidx_hbm, out_hbm, idx_v, rows_v, out_v, sem):
    wid = _worker_id()
    base = wid * _CHUNK_A
    pltpu.sync_copy(hidx_hbm.at[pl.ds(2 * base, 2 * _CHUNK_A)], idx_v)
    pltpu.async_copy(tab_hbm.at[idx_v], rows_v, sem).wait()

    def body(i, carry):
        lanes = lax.iota(jnp.int32, 16) + i * 16
        f0 = plsc.load_gather(rows_v, [lanes * 2])
        f1 = plsc.load_gather(rows_v, [lanes * 2 + 1])
        out_v[pl.ds(i * 16, 16)] = _pack_bf16_pair(f0, f1)
        return carry

    lax.fori_loop(0, _CHUNK_A // 16, body, 0)
    pltpu.sync_copy(out_v, out_hbm.at[pl.ds(base, _CHUNK_A)])


_NSLICES = 2
_NSL = _N // _NSLICES
_NPW = _NSL // _NW
_CB = 2048
_NCH = _NPW // _CB


def _make_encode(slice_off):
    @functools.partial(
        pl.kernel,
        out_type=jax.ShapeDtypeStruct((_N_LEVELS, _NSL), jnp.int32),
        mesh=_MESH,
        scratch_types=[
            pltpu.VMEM((_GTOT,), jnp.int32),
            pltpu.VMEM((_CB,), jnp.float32),
            pltpu.VMEM((_CB,), jnp.float32),
            pltpu.VMEM((_CB,), jnp.float32),
            pltpu.VMEM((_CB * _N_LEVELS,), jnp.int32),
        ],
        compiler_params=pltpu.CompilerParams(needs_layout_passes=False),
    )
    def _encode(x_hbm, grid_hbm, pe_hbm, grid_v, xb0, xb1, xb2, peb):
        wid = _worker_id()
        base = wid * _NPW
        pltpu.sync_copy(grid_hbm, grid_v)

        def chunk_body(ci, carry):
            off = base + ci * _CB
            g_off = slice_off + off
            pltpu.sync_copy(x_hbm.at[pl.ds(g_off, _CB)], xb0)
            pltpu.sync_copy(x_hbm.at[pl.ds(_N + g_off, _CB)], xb1)
            pltpu.sync_copy(x_hbm.at[pl.ds(2 * _N + g_off, _CB)], xb2)

            def g_body(g, gcarry):
                p = g * 16
                x0 = xb0[pl.ds(p, 16)]
                x1 = xb1[pl.ds(p, 16)]
                x2 = xb2[pl.ds(p, 16)]
                for l in range(_N_LEVELS):
                    res = _RES[l]
                    s = _SIDES[l]
                    rf = jnp.float32(res)
                    s0 = x0 * rf
                    s1 = x1 * rf
                    s2 = x2 * rf
                    i0 = s0.astype(jnp.int32)
                    i1 = s1.astype(jnp.int32)
                    i2 = s2.astype(jnp.int32)
                    f0 = s0 - i0.astype(jnp.float32)
                    f1 = s1 - i1.astype(jnp.float32)
                    f2 = s2 - i2.astype(jnp.float32)
                    t0 = f0 * f0 * (3.0 - 2.0 * f0)
                    t1 = f1 * f1 * (3.0 - 2.0 * f1)
                    t2 = f2 * f2 * (3.0 - 2.0 * f2)
                    flat = (i0 * s + i1) * s + i2 + _LEVEL_OFFS[l]

                    tw0 = plsc.pack(t0, t0, format=plsc.PackFormat.INTERLEAVED)
                    tw1 = plsc.pack(t1, t1, format=plsc.PackFormat.INTERLEAVED)
                    tw2 = plsc.pack(t2, t2, format=plsc.PackFormat.INTERLEAVED)

                    def corner(d):
                        w = plsc.load_gather(grid_v, [flat + d])
                        return plsc.bitcast(w, jnp.bfloat16)

                    c000 = corner(0)
                    c001 = corner(1)
                    c010 = corner(s)
                    c011 = corner(s + 1)
                    c100 = corner(s * s)
                    c101 = corner(s * s + 1)
                    c110 = corner(s * s + s)
                    c111 = corner(s * s + s + 1)

                    def lerp(a, b, t):
                        return a + t * (b - a)

                    m00 = lerp(c000, c001, tw2)
                    m01 = lerp(c010, c011, tw2)
                    m10 = lerp(c100, c101, tw2)
                    m11 = lerp(c110, c111, tw2)
                    n0 = lerp(m00, m01, tw1)
                    n1 = lerp(m10, m11, tw1)
                    acc = lerp(n0, n1, tw0)

                    peb[pl.ds(l * _CB + p, 16)] = plsc.bitcast(acc, jnp.int32)
                return gcarry

            lax.fori_loop(0, _CB // 16, g_body, 0)
            for l in range(_N_LEVELS):
                pltpu.sync_copy(peb.at[pl.ds(l * _CB, _CB)],
                                pe_hbm.at[l, pl.ds(off, _CB)])
            return carry

        lax.fori_loop(0, _NCH, chunk_body, 0)

    return _encode


_ENCODERS = [_make_encode(s * _NSL) for s in range(_NSLICES)]


_BLK = 4096


def _contract0(a, b):
    return lax.dot_general(a, b, (((0,), (0,)), ((), ())),
                           preferred_element_type=jnp.float32)


def _mlp_body(pe_ref, e_ref, x_ref, w1lo_ref, w1hi_ref, w1b_ref, w2_ref, w3_ref,
              b1_ref, b2_ref, b3_ref, o_ref):
    w = pe_ref[...]
    lo = lax.bitcast_convert_type(lax.shift_left(w, 16), jnp.float32)
    hi = lax.bitcast_convert_type(w & jnp.int32(-65536), jnp.float32)
    eb = e_ref[...].astype(jnp.bfloat16)
    h = _contract0(w1lo_ref[...], lo.astype(jnp.bfloat16))
    h = h + _contract0(w1hi_ref[...], hi.astype(jnp.bfloat16))
    h = h + _contract0(w1b_ref[...], eb)
    h = jnp.tanh(h + b1_ref[...])
    h2 = _contract0(w2_ref[...], h.astype(jnp.bfloat16))
    h2 = jnp.tanh(h2 + b2_ref[...])
    o = _contract0(w3_ref[...], h2.astype(jnp.bfloat16))
    o = o + b3_ref[...]
    xn = (x_ref[...] - _BB0) * (1.0 / (_BB1 - _BB0))
    o_ref[...] = (o + xn) * (_BB1 - _BB0) + _BB0


def _mlp(pe_t, e_t, x_t, W1lo, W1hi, W1b, W2b, W3b, b1r, b2r, b3r, blk_off):
    grid = (_NSL // _BLK,)
    return pl.pallas_call(
        _mlp_body,
        grid=grid,
        in_specs=[
            pl.BlockSpec((_N_LEVELS, _BLK), lambda i: (0, i)),
            pl.BlockSpec((_N_FEAT_E, _BLK), lambda i: (0, i + blk_off)),
            pl.BlockSpec((3, _BLK), lambda i: (0, i + blk_off)),
            pl.BlockSpec((_N_LEVELS, _WIDTH), lambda i: (0, 0)),
            pl.BlockSpec((_N_LEVELS, _WIDTH), lambda i: (0, 0)),
            pl.BlockSpec((_N_FEAT_E, _WIDTH), lambda i: (0, 0)),
            pl.BlockSpec((_WIDTH, _WIDTH), lambda i: (0, 0)),
            pl.BlockSpec((_WIDTH, 3), lambda i: (0, 0)),
            pl.BlockSpec((_WIDTH, 1), lambda i: (0, 0)),
            pl.BlockSpec((_WIDTH, 1), lambda i: (0, 0)),
            pl.BlockSpec((3, 1), lambda i: (0, 0)),
        ],
        out_specs=pl.BlockSpec((3, _BLK), lambda i: (0, i)),
        out_shape=jax.ShapeDtypeStruct((3, _NSL), jnp.float32),
    )(pe_t, e_t, x_t, W1lo, W1hi, W1b, W2b, W3b, b1r, b2r, b3r)


def kernel(x, e, tables, W1, b1, W2, b2, W3, b3):
    x_t = x.T
    x_flat = x_t.reshape(3 * _N)
    e_t = e.T
    tab_flat = jnp.transpose(tables, (0, 2, 1)).reshape(_N_LEVELS * 2 * _T)
    hidx = jnp.asarray(_HIDX2)

    grid_packed = _compact(tab_flat, hidx)

    W1lo = W1[0:2 * _N_LEVELS:2].astype(jnp.bfloat16)
    W1hi = W1[1:2 * _N_LEVELS:2].astype(jnp.bfloat16)
    W1b = W1[2 * _N_LEVELS:].astype(jnp.bfloat16)
    W2b = W2.astype(jnp.bfloat16)
    W3b = W3.astype(jnp.bfloat16)
    b1r = b1.reshape(_WIDTH, 1)
    b2r = b2.reshape(_WIDTH, 1)
    b3r = b3.reshape(3, 1)

    parts = []
    for s in range(_NSLICES):
        pe_s = _ENCODERS[s](x_flat, grid_packed)
        parts.append(_mlp(pe_s, e_t, x_t, W1lo, W1hi, W1b, W2b, W3b,
                          b1r, b2r, b3r, s * (_NSL // _BLK)))
    out_t = jnp.concatenate(parts, axis=1)
    return out_t.T.reshape(x.shape)


# speedup vs baseline: 31.5767x; 1.0329x over previous
import functools

import jax
import jax.numpy as jnp
import numpy as np
from jax import lax
from jax.experimental import pallas as pl
from jax.experimental.pallas import tpu as pltpu
from jax.experimental.pallas import tpu_sc as plsc

_BB0 = 0.0
_BB1 = 1.0
_BASE_RES = 8
_N_LEVELS = 6
_LEVEL_SCALE = 1.32
_LOG2_T = 21
_T = 2 ** _LOG2_T
_RES = [int(np.floor(_BASE_RES * _LEVEL_SCALE ** l)) for l in range(_N_LEVELS)]
_SIDES = [r + 1 for r in _RES]
_N = 1048576
_WIDTH = 64
_N_FEAT_E = 16

_NC = 2
_NS = 16
_NW = _NC * _NS

_LEVEL_OFFS = np.cumsum([0] + [s ** 3 for s in _SIDES]).tolist()
_GTOT_RAW = _LEVEL_OFFS[-1]
_CHUNK_A = -(-_GTOT_RAW // (_NW * 16)) * 16
_GTOT = _CHUNK_A * _NW


def _build_hidx():
    hidx2 = np.zeros(2 * _GTOT, dtype=np.int32)
    for l, s in enumerate(_SIDES):
        ii, jj, kk = np.meshgrid(np.arange(s, dtype=np.uint32),
                                 np.arange(s, dtype=np.uint32),
                                 np.arange(s, dtype=np.uint32), indexing="ij")
        h = (ii * np.uint32(1)) ^ (jj * np.uint32(2654435761)) ^ (kk * np.uint32(805459861))
        h &= np.uint32(_T - 1)
        h = h.ravel().astype(np.int64)
        hidx2[2 * _LEVEL_OFFS[l]:2 * _LEVEL_OFFS[l + 1]:2] = (2 * l * _T + h).astype(np.int32)
        hidx2[2 * _LEVEL_OFFS[l] + 1:2 * _LEVEL_OFFS[l + 1]:2] = ((2 * l + 1) * _T + h).astype(np.int32)
    return hidx2


_HIDX2 = _build_hidx()

_MESH = plsc.VectorSubcoreMesh(core_axis_name="c", subcore_axis_name="s")


def _worker_id():
    return lax.axis_index("c") * _NS + lax.axis_index("s")


def _pack_bf16_pair(f0, f1):
    a0 = plsc.bitcast(f0, jnp.int32)
    a1 = plsc.bitcast(f1, jnp.int32)
    lo = lax.shift_right_logical(a0 + 0x8000, 16)
    hi = (a1 + 0x8000) & jnp.int32(-65536)
    return lo | hi


def _unpack_bf16_pair(w):
    f0 = plsc.bitcast(lax.shift_left(w, 16), jnp.float32)
    f1 = plsc.bitcast(w, jnp.float32)
    return f0, f1


@functools.partial(
    pl.kernel,
    out_type=jax.ShapeDtypeStruct((_GTOT,), jnp.int32),
    mesh=_MESH,
    scratch_types=[
        pltpu.VMEM((2 * _CHUNK_A,), jnp.int32),
        pltpu.VMEM((2 * _CHUNK_A,), jnp.float32),
        pltpu.VMEM((_CHUNK_A,), jnp.int32),
        pltpu.SemaphoreType.DMA,
    ],
    compiler_params=pltpu.CompilerParams(needs_layout_passes=False),
)
def _compact(tab_hbm, hidx_hbm, out_hbm, idx_v, rows_v, out_v, sem):
    wid = _worker_id()
    base = wid * _CHUNK_A
    pltpu.sync_copy(hidx_hbm.at[pl.ds(2 * base, 2 * _CHUNK_A)], idx_v)
    pltpu.async_copy(tab_hbm.at[idx_v], rows_v, sem).wait()

    def body(i, carry):
        lanes = lax.iota(jnp.int32, 16) + i * 16
        f0 = plsc.load_gather(rows_v, [lanes * 2])
        f1 = plsc.load_gather(rows_v, [lanes * 2 + 1])
        out_v[pl.ds(i * 16, 16)] = _pack_bf16_pair(f0, f1)
        return carry

    lax.fori_loop(0, _CHUNK_A // 16, body, 0)
    pltpu.sync_copy(out_v, out_hbm.at[pl.ds(base, _CHUNK_A)])


_NSLICES = 4
_NSL = _N // _NSLICES
_NPW = _NSL // _NW
_CB = 2048
_NCH = _NPW // _CB


def _make_encode(slice_off):
    @functools.partial(
        pl.kernel,
        out_type=jax.ShapeDtypeStruct((_N_LEVELS, _NSL), jnp.int32),
        mesh=_MESH,
        scratch_types=[
            pltpu.VMEM((_GTOT,), jnp.int32),
            pltpu.VMEM((_CB,), jnp.float32),
            pltpu.VMEM((_CB,), jnp.float32),
            pltpu.VMEM((_CB,), jnp.float32),
            pltpu.VMEM((_CB * _N_LEVELS,), jnp.int32),
        ],
        compiler_params=pltpu.CompilerParams(needs_layout_passes=False),
    )
    def _encode(x_hbm, grid_hbm, pe_hbm, grid_v, xb0, xb1, xb2, peb):
        wid = _worker_id()
        base = wid * _NPW
        pltpu.sync_copy(grid_hbm, grid_v)

        def chunk_body(ci, carry):
            off = base + ci * _CB
            g_off = slice_off + off
            pltpu.sync_copy(x_hbm.at[pl.ds(g_off, _CB)], xb0)
            pltpu.sync_copy(x_hbm.at[pl.ds(_N + g_off, _CB)], xb1)
            pltpu.sync_copy(x_hbm.at[pl.ds(2 * _N + g_off, _CB)], xb2)

            def g_body(g, gcarry):
                p = g * 16
                x0 = xb0[pl.ds(p, 16)]
                x1 = xb1[pl.ds(p, 16)]
                x2 = xb2[pl.ds(p, 16)]
                for l in range(_N_LEVELS):
                    res = _RES[l]
                    s = _SIDES[l]
                    rf = jnp.float32(res)
                    s0 = x0 * rf
                    s1 = x1 * rf
                    s2 = x2 * rf
                    i0 = s0.astype(jnp.int32)
                    i1 = s1.astype(jnp.int32)
                    i2 = s2.astype(jnp.int32)
                    f0 = s0 - i0.astype(jnp.float32)
                    f1 = s1 - i1.astype(jnp.float32)
                    f2 = s2 - i2.astype(jnp.float32)
                    t0 = f0 * f0 * (3.0 - 2.0 * f0)
                    t1 = f1 * f1 * (3.0 - 2.0 * f1)
                    t2 = f2 * f2 * (3.0 - 2.0 * f2)
                    flat = (i0 * s + i1) * s + i2 + _LEVEL_OFFS[l]

                    tw0 = plsc.pack(t0, t0, format=plsc.PackFormat.INTERLEAVED)
                    tw1 = plsc.pack(t1, t1, format=plsc.PackFormat.INTERLEAVED)
                    tw2 = plsc.pack(t2, t2, format=plsc.PackFormat.INTERLEAVED)

                    def corner(d):
                        w = plsc.load_gather(grid_v, [flat + d])
                        return plsc.bitcast(w, jnp.bfloat16)

                    c000 = corner(0)
                    c001 = corner(1)
                    c010 = corner(s)
                    c011 = corner(s + 1)
                    c100 = corner(s * s)
                    c101 = corner(s * s + 1)
                    c110 = corner(s * s + s)
                    c111 = corner(s * s + s + 1)

                    def lerp(a, b, t):
                        return a + t * (b - a)

                    m00 = lerp(c000, c001, tw2)
                    m01 = lerp(c010, c011, tw2)
                    m10 = lerp(c100, c101, tw2)
                    m11 = lerp(c110, c111, tw2)
                    n0 = lerp(m00, m01, tw1)
                    n1 = lerp(m10, m11, tw1)
                    acc = lerp(n0, n1, tw0)

                    peb[pl.ds(l * _CB + p, 16)] = plsc.bitcast(acc, jnp.int32)
                return gcarry

            lax.fori_loop(0, _CB // 16, g_body, 0)
            for l in range(_N_LEVELS):
                pltpu.sync_copy(peb.at[pl.ds(l * _CB, _CB)],
                                pe_hbm.at[l, pl.ds(off, _CB)])
            return carry

        lax.fori_loop(0, _NCH, chunk_body, 0)

    return _encode


_ENCODERS = [_make_encode(s * _NSL) for s in range(_NSLICES)]


_BLK = 4096


def _contract0(a, b):
    return lax.dot_general(a, b, (((0,), (0,)), ((), ())),
                           preferred_element_type=jnp.float32)


def _mlp_body(pe_ref, e_ref, x_ref, w1lo_ref, w1hi_ref, w1b_ref, w2_ref, w3_ref,
              b1_ref, b2_ref, b3_ref, o_ref):
    w = pe_ref[...]
    lo = lax.bitcast_convert_type(lax.shift_left(w, 16), jnp.float32)
    hi = lax.bitcast_convert_type(w & jnp.int32(-65536), jnp.float32)
    eb = e_ref[...].astype(jnp.bfloat16)
    h = _contract0(w1lo_ref[...], lo.astype(jnp.bfloat16))
    h = h + _contract0(w1hi_ref[...], hi.astype(jnp.bfloat16))
    h = h + _contract0(w1b_ref[...], eb)
    h = jnp.tanh(h + b1_ref[...])
    h2 = _contract0(w2_ref[...], h.astype(jnp.bfloat16))
    h2 = jnp.tanh(h2 + b2_ref[...])
    o = _contract0(w3_ref[...], h2.astype(jnp.bfloat16))
    o = o + b3_ref[...]
    xn = (x_ref[...] - _BB0) * (1.0 / (_BB1 - _BB0))
    o_ref[...] = (o + xn) * (_BB1 - _BB0) + _BB0


def _mlp(pe_t, e_t, x_t, W1lo, W1hi, W1b, W2b, W3b, b1r, b2r, b3r, blk_off):
    grid = (_NSL // _BLK,)
    return pl.pallas_call(
        _mlp_body,
        grid=grid,
        in_specs=[
            pl.BlockSpec((_N_LEVELS, _BLK), lambda i: (0, i)),
            pl.BlockSpec((_N_FEAT_E, _BLK), lambda i: (0, i + blk_off)),
            pl.BlockSpec((3, _BLK), lambda i: (0, i + blk_off)),
            pl.BlockSpec((_N_LEVELS, _WIDTH), lambda i: (0, 0)),
            pl.BlockSpec((_N_LEVELS, _WIDTH), lambda i: (0, 0)),
            pl.BlockSpec((_N_FEAT_E, _WIDTH), lambda i: (0, 0)),
            pl.BlockSpec((_WIDTH, _WIDTH), lambda i: (0, 0)),
            pl.BlockSpec((_WIDTH, 3), lambda i: (0, 0)),
            pl.BlockSpec((_WIDTH, 1), lambda i: (0, 0)),
            pl.BlockSpec((_WIDTH, 1), lambda i: (0, 0)),
            pl.BlockSpec((3, 1), lambda i: (0, 0)),
        ],
        out_specs=pl.BlockSpec((3, _BLK), lambda i: (0, i)),
        out_shape=jax.ShapeDtypeStruct((3, _NSL), jnp.float32),
    )(pe_t, e_t, x_t, W1lo, W1hi, W1b, W2b, W3b, b1r, b2r, b3r)


def kernel(x, e, tables, W1, b1, W2, b2, W3, b3):
    x_t = x.T
    x_flat = lax.reshape(x, (3 * _N,), dimensions=(1, 0))
    e_t = e.T
    tab_flat = jnp.transpose(tables, (0, 2, 1)).reshape(_N_LEVELS * 2 * _T)
    hidx = jnp.asarray(_HIDX2)

    grid_packed = _compact(tab_flat, hidx)

    W1lo = W1[0:2 * _N_LEVELS:2].astype(jnp.bfloat16)
    W1hi = W1[1:2 * _N_LEVELS:2].astype(jnp.bfloat16)
    W1b = W1[2 * _N_LEVELS:].astype(jnp.bfloat16)
    W2b = W2.astype(jnp.bfloat16)
    W3b = W3.astype(jnp.bfloat16)
    b1r = b1.reshape(_WIDTH, 1)
    b2r = b2.reshape(_WIDTH, 1)
    b3r = b3.reshape(3, 1)

    parts = []
    for s in range(_NSLICES):
        pe_s = _ENCODERS[s](x_flat, grid_packed)
        parts.append(_mlp(pe_s, e_t, x_t, W1lo, W1hi, W1b, W2b, W3b,
                          b1r, b2r, b3r, s * (_NSL // _BLK)))
    out_t = jnp.concatenate(parts, axis=1)
    return out_t.T.reshape(x.shape)


# speedup vs baseline: 33.6865x; 1.0668x over previous
import functools

import jax
import jax.numpy as jnp
import numpy as np
from jax import lax
from jax.experimental import pallas as pl
from jax.experimental.pallas import tpu as pltpu
from jax.experimental.pallas import tpu_sc as plsc

_BB0 = 0.0
_BB1 = 1.0
_BASE_RES = 8
_N_LEVELS = 6
_LEVEL_SCALE = 1.32
_LOG2_T = 21
_T = 2 ** _LOG2_T
_RES = [int(np.floor(_BASE_RES * _LEVEL_SCALE ** l)) for l in range(_N_LEVELS)]
_SIDES = [r + 1 for r in _RES]
_N = 1048576
_WIDTH = 64
_N_FEAT_E = 16

_NC = 2
_NS = 16
_NW = _NC * _NS

_LEVEL_OFFS = np.cumsum([0] + [s ** 3 for s in _SIDES]).tolist()
_GTOT_RAW = _LEVEL_OFFS[-1]
_CHUNK_A = -(-_GTOT_RAW // (_NW * 16)) * 16
_GTOT = _CHUNK_A * _NW


def _build_hidx():
    hidx2 = np.zeros(2 * _GTOT, dtype=np.int32)
    for l, s in enumerate(_SIDES):
        ii, jj, kk = np.meshgrid(np.arange(s, dtype=np.uint32),
                                 np.arange(s, dtype=np.uint32),
                                 np.arange(s, dtype=np.uint32), indexing="ij")
        h = (ii * np.uint32(1)) ^ (jj * np.uint32(2654435761)) ^ (kk * np.uint32(805459861))
        h &= np.uint32(_T - 1)
        h = h.ravel().astype(np.int64)
        hidx2[2 * _LEVEL_OFFS[l]:2 * _LEVEL_OFFS[l + 1]:2] = (2 * l * _T + h).astype(np.int32)
        hidx2[2 * _LEVEL_OFFS[l] + 1:2 * _LEVEL_OFFS[l + 1]:2] = ((2 * l + 1) * _T + h).astype(np.int32)
    return hidx2


_HIDX2 = _build_hidx()

_MESH = plsc.VectorSubcoreMesh(core_axis_name="c", subcore_axis_name="s")


def _worker_id():
    return lax.axis_index("c") * _NS + lax.axis_index("s")


def _pack_bf16_pair(f0, f1):
    a0 = plsc.bitcast(f0, jnp.int32)
    a1 = plsc.bitcast(f1, jnp.int32)
    lo = lax.shift_right_logical(a0 + 0x8000, 16)
    hi = (a1 + 0x8000) & jnp.int32(-65536)
    return lo | hi


def _unpack_bf16_pair(w):
    f0 = plsc.bitcast(lax.shift_left(w, 16), jnp.float32)
    f1 = plsc.bitcast(w, jnp.float32)
    return f0, f1


@functools.partial(
    pl.kernel,
    out_type=jax.ShapeDtypeStruct((_GTOT,), jnp.int32),
    mesh=_MESH,
    scratch_types=[
        pltpu.VMEM((2 * _CHUNK_A,), jnp.int32),
        pltpu.VMEM((2 * _CHUNK_A,), jnp.float32),
        pltpu.VMEM((_CHUNK_A,), jnp.int32),
        pltpu.SemaphoreType.DMA,
    ],
    compiler_params=pltpu.CompilerParams(needs_layout_passes=False),
)
def _compact(tab_hbm, hidx_hbm, out_hbm, idx_v, rows_v, out_v, sem):
    wid = _worker_id()
    base = wid * _CHUNK_A
    pltpu.sync_copy(hidx_hbm.at[pl.ds(2 * base, 2 * _CHUNK_A)], idx_v)
    pltpu.async_copy(tab_hbm.at[idx_v], rows_v, sem).wait()

    def body(i, carry):
        lanes = lax.iota(jnp.int32, 16) + i * 16
        f0 = plsc.load_gather(rows_v, [lanes * 2])
        f1 = plsc.load_gather(rows_v, [lanes * 2 + 1])
        out_v[pl.ds(i * 16, 16)] = _pack_bf16_pair(f0, f1)
        return carry

    lax.fori_loop(0, _CHUNK_A // 16, body, 0)
    pltpu.sync_copy(out_v, out_hbm.at[pl.ds(base, _CHUNK_A)])


_NSLICES = 4
_NSL = _N // _NSLICES
_NPW = _NSL // _NW
_CB = 2048
_NCH = _NPW // _CB


def _make_encode(slice_off):
    @functools.partial(
        pl.kernel,
        out_type=jax.ShapeDtypeStruct((_N_LEVELS, _NSL), jnp.int32),
        mesh=_MESH,
        scratch_types=[
            pltpu.VMEM((_GTOT,), jnp.int32),
            pltpu.VMEM((_CB,), jnp.float32),
            pltpu.VMEM((_CB,), jnp.float32),
            pltpu.VMEM((_CB,), jnp.float32),
            pltpu.VMEM((_CB,), jnp.float32),
            pltpu.VMEM((_CB,), jnp.float32),
            pltpu.VMEM((_CB,), jnp.float32),
            pltpu.VMEM((_CB * _N_LEVELS,), jnp.int32),
            pltpu.VMEM((_CB * _N_LEVELS,), jnp.int32),
            pltpu.SemaphoreType.DMA,
            pltpu.SemaphoreType.DMA,
            pltpu.SemaphoreType.DMA,
            pltpu.SemaphoreType.DMA,
        ],
        compiler_params=pltpu.CompilerParams(needs_layout_passes=False),
    )
    def _encode(x_hbm, grid_hbm, pe_hbm, grid_v,
                xb00, xb01, xb02, xb10, xb11, xb12, pea, pebb,
                sx0, sx1, sp0, sp1):
        wid = _worker_id()
        base = wid * _NPW
        xbufs = [(xb00, xb01, xb02), (xb10, xb11, xb12)]
        pebufs = [pea, pebb]
        sx = [sx0, sx1]
        sp = [sp0, sp1]

        def issue_x(ci, b):
            g_off = slice_off + base + ci * _CB
            for c in range(3):
                pltpu.async_copy(x_hbm.at[pl.ds(c * _N + g_off, _CB)],
                                 xbufs[b][c], sx[b])

        def wait_x(b):
            for c in range(3):
                pltpu.make_async_copy(x_hbm.at[pl.ds(0, _CB)],
                                      xbufs[b][c], sx[b]).wait()

        def issue_pe(ci, b):
            off = base + ci * _CB
            for l in range(_N_LEVELS):
                pltpu.async_copy(pebufs[b].at[pl.ds(l * _CB, _CB)],
                                 pe_hbm.at[l, pl.ds(off, _CB)], sp[b])

        def drain_pe(b):
            for l in range(_N_LEVELS):
                pltpu.make_async_copy(pebufs[b].at[pl.ds(l * _CB, _CB)],
                                      pe_hbm.at[l, pl.ds(base, _CB)], sp[b]).wait()

        issue_x(0, 0)
        pltpu.sync_copy(grid_hbm, grid_v)

        def chunk_pair(ci2, carry):
            for b in range(2):
                ci = ci2 * 2 + b
                peb = pebufs[b]
                xb0, xb1, xb2 = xbufs[b]
                wait_x(b)

                @pl.when(ci + 1 < _NCH)
                def _():
                    issue_x(ci + 1, 1 - b)

                @pl.when(ci >= 2)
                def _():
                    drain_pe(b)

                def g_body(g, gcarry):
                    p = g * 16
                    x0 = xb0[pl.ds(p, 16)]
                    x1 = xb1[pl.ds(p, 16)]
                    x2 = xb2[pl.ds(p, 16)]
                    for l in range(_N_LEVELS):
                        res = _RES[l]
                        s = _SIDES[l]
                        rf = jnp.float32(res)
                        s0 = x0 * rf
                        s1 = x1 * rf
                        s2 = x2 * rf
                        i0 = s0.astype(jnp.int32)
                        i1 = s1.astype(jnp.int32)
                        i2 = s2.astype(jnp.int32)
                        f0 = s0 - i0.astype(jnp.float32)
                        f1 = s1 - i1.astype(jnp.float32)
                        f2 = s2 - i2.astype(jnp.float32)
                        t0 = f0 * f0 * (3.0 - 2.0 * f0)
                        t1 = f1 * f1 * (3.0 - 2.0 * f1)
                        t2 = f2 * f2 * (3.0 - 2.0 * f2)
                        flat = (i0 * s + i1) * s + i2 + _LEVEL_OFFS[l]

                        tw0 = plsc.pack(t0, t0, format=plsc.PackFormat.INTERLEAVED)
                        tw1 = plsc.pack(t1, t1, format=plsc.PackFormat.INTERLEAVED)
                        tw2 = plsc.pack(t2, t2, format=plsc.PackFormat.INTERLEAVED)

                        def corner(d):
                            w = plsc.load_gather(grid_v, [flat + d])
                            return plsc.bitcast(w, jnp.bfloat16)

                        c000 = corner(0)
                        c001 = corner(1)
                        c010 = corner(s)
                        c011 = corner(s + 1)
                        c100 = corner(s * s)
                        c101 = corner(s * s + 1)
                        c110 = corner(s * s + s)
                        c111 = corner(s * s + s + 1)

                        def lerp(u, v, t):
                            return u + t * (v - u)

                        m00 = lerp(c000, c001, tw2)
                        m01 = lerp(c010, c011, tw2)
                        m10 = lerp(c100, c101, tw2)
                        m11 = lerp(c110, c111, tw2)
                        n0 = lerp(m00, m01, tw1)
                        n1 = lerp(m10, m11, tw1)
                        acc = lerp(n0, n1, tw0)

                        peb[pl.ds(l * _CB + p, 16)] = plsc.bitcast(acc, jnp.int32)
                    return gcarry

                lax.fori_loop(0, _CB // 16, g_body, 0)
                issue_pe(ci, b)
            return carry

        lax.fori_loop(0, _NCH // 2, chunk_pair, 0)
        drain_pe(0)
        drain_pe(1)

    return _encode


_ENCODERS = [_make_encode(s * _NSL) for s in range(_NSLICES)]


_BLK = 4096


def _contract0(a, b):
    return lax.dot_general(a, b, (((0,), (0,)), ((), ())),
                           preferred_element_type=jnp.float32)


def _mlp_body(pe_ref, e_ref, x_ref, w1lo_ref, w1hi_ref, w1b_ref, w2_ref, w3_ref,
              b1_ref, b2_ref, b3_ref, o_ref):
    w = pe_ref[...]
    lo = lax.bitcast_convert_type(lax.shift_left(w, 16), jnp.float32)
    hi = lax.bitcast_convert_type(w & jnp.int32(-65536), jnp.float32)
    eb = e_ref[...].astype(jnp.bfloat16)
    h = _contract0(w1lo_ref[...], lo.astype(jnp.bfloat16))
    h = h + _contract0(w1hi_ref[...], hi.astype(jnp.bfloat16))
    h = h + _contract0(w1b_ref[...], eb)
    h = jnp.tanh(h + b1_ref[...])
    h2 = _contract0(w2_ref[...], h.astype(jnp.bfloat16))
    h2 = jnp.tanh(h2 + b2_ref[...])
    o = _contract0(w3_ref[...], h2.astype(jnp.bfloat16))
    o = o + b3_ref[...]
    xn = (x_ref[...] - _BB0) * (1.0 / (_BB1 - _BB0))
    o_ref[...] = (o + xn) * (_BB1 - _BB0) + _BB0


def _mlp(pe_t, e_t, x_t, W1lo, W1hi, W1b, W2b, W3b, b1r, b2r, b3r, blk_off):
    grid = (_NSL // _BLK,)
    return pl.pallas_call(
        _mlp_body,
        grid=grid,
        in_specs=[
            pl.BlockSpec((_N_LEVELS, _BLK), lambda i: (0, i)),
            pl.BlockSpec((_N_FEAT_E, _BLK), lambda i: (0, i + blk_off)),
            pl.BlockSpec((3, _BLK), lambda i: (0, i + blk_off)),
            pl.BlockSpec((_N_LEVELS, _WIDTH), lambda i: (0, 0)),
            pl.BlockSpec((_N_LEVELS, _WIDTH), lambda i: (0, 0)),
            pl.BlockSpec((_N_FEAT_E, _WIDTH), lambda i: (0, 0)),
            pl.BlockSpec((_WIDTH, _WIDTH), lambda i: (0, 0)),
            pl.BlockSpec((_WIDTH, 3), lambda i: (0, 0)),
            pl.BlockSpec((_WIDTH, 1), lambda i: (0, 0)),
            pl.BlockSpec((_WIDTH, 1), lambda i: (0, 0)),
            pl.BlockSpec((3, 1), lambda i: (0, 0)),
        ],
        out_specs=pl.BlockSpec((3, _BLK), lambda i: (0, i)),
        out_shape=jax.ShapeDtypeStruct((3, _NSL), jnp.float32),
    )(pe_t, e_t, x_t, W1lo, W1hi, W1b, W2b, W3b, b1r, b2r, b3r)


def kernel(x, e, tables, W1, b1, W2, b2, W3, b3):
    x_t = x.T
    x_flat = lax.reshape(x, (3 * _N,), dimensions=(1, 0))
    e_t = e.T
    tab_flat = jnp.transpose(tables, (0, 2, 1)).reshape(_N_LEVELS * 2 * _T)
    hidx = jnp.asarray(_HIDX2)

    grid_packed = _compact(tab_flat, hidx)

    W1lo = W1[0:2 * _N_LEVELS:2].astype(jnp.bfloat16)
    W1hi = W1[1:2 * _N_LEVELS:2].astype(jnp.bfloat16)
    W1b = W1[2 * _N_LEVELS:].astype(jnp.bfloat16)
    W2b = W2.astype(jnp.bfloat16)
    W3b = W3.astype(jnp.bfloat16)
    b1r = b1.reshape(_WIDTH, 1)
    b2r = b2.reshape(_WIDTH, 1)
    b3r = b3.reshape(3, 1)

    parts = []
    for s in range(_NSLICES):
        pe_s = _ENCODERS[s](x_flat, grid_packed)
        parts.append(_mlp(pe_s, e_t, x_t, W1lo, W1hi, W1b, W2b, W3b,
                          b1r, b2r, b3r, s * (_NSL // _BLK)))
    out_t = jnp.concatenate(parts, axis=1)
    return out_t.T.reshape(x.shape)


# speedup vs baseline: 38.7748x; 1.1511x over previous
import functools

import jax
import jax.numpy as jnp
import numpy as np
from jax import lax
from jax.experimental import pallas as pl
from jax.experimental.pallas import tpu as pltpu
from jax.experimental.pallas import tpu_sc as plsc

_BB0 = 0.0
_BB1 = 1.0
_BASE_RES = 8
_N_LEVELS = 6
_LEVEL_SCALE = 1.32
_LOG2_T = 21
_T = 2 ** _LOG2_T
_RES = [int(np.floor(_BASE_RES * _LEVEL_SCALE ** l)) for l in range(_N_LEVELS)]
_SIDES = [r + 1 for r in _RES]
_N = 1048576
_WIDTH = 64
_N_FEAT_E = 16

_NC = 2
_NS = 16
_NW = _NC * _NS

_LEVEL_OFFS = np.cumsum([0] + [s ** 3 for s in _SIDES]).tolist()
_GTOT_RAW = _LEVEL_OFFS[-1]
_CHUNK_A = -(-_GTOT_RAW // (_NW * 16)) * 16
_GTOT = _CHUNK_A * _NW


def _build_hidx():
    hidx2 = np.zeros(2 * _GTOT, dtype=np.int32)
    for l, s in enumerate(_SIDES):
        ii, jj, kk = np.meshgrid(np.arange(s, dtype=np.uint32),
                                 np.arange(s, dtype=np.uint32),
                                 np.arange(s, dtype=np.uint32), indexing="ij")
        h = (ii * np.uint32(1)) ^ (jj * np.uint32(2654435761)) ^ (kk * np.uint32(805459861))
        h &= np.uint32(_T - 1)
        h = h.ravel().astype(np.int64)
        hidx2[2 * _LEVEL_OFFS[l]:2 * _LEVEL_OFFS[l + 1]:2] = (2 * l * _T + h).astype(np.int32)
        hidx2[2 * _LEVEL_OFFS[l] + 1:2 * _LEVEL_OFFS[l + 1]:2] = ((2 * l + 1) * _T + h).astype(np.int32)
    return hidx2


_HIDX2 = _build_hidx()

_MESH = plsc.VectorSubcoreMesh(core_axis_name="c", subcore_axis_name="s")


def _worker_id():
    return lax.axis_index("c") * _NS + lax.axis_index("s")


def _pack_bf16_pair(f0, f1):
    a0 = plsc.bitcast(f0, jnp.int32)
    a1 = plsc.bitcast(f1, jnp.int32)
    lo = lax.shift_right_logical(a0 + 0x8000, 16)
    hi = (a1 + 0x8000) & jnp.int32(-65536)
    return lo | hi


def _unpack_bf16_pair(w):
    f0 = plsc.bitcast(lax.shift_left(w, 16), jnp.float32)
    f1 = plsc.bitcast(w, jnp.float32)
    return f0, f1


@functools.partial(
    pl.kernel,
    out_type=jax.ShapeDtypeStruct((_GTOT,), jnp.int32),
    mesh=_MESH,
    scratch_types=[
        pltpu.VMEM((2 * _CHUNK_A,), jnp.int32),
        pltpu.VMEM((2 * _CHUNK_A,), jnp.float32),
        pltpu.VMEM((_CHUNK_A,), jnp.int32),
        pltpu.SemaphoreType.DMA,
    ],
    compiler_params=pltpu.CompilerParams(needs_layout_passes=False),
)
def _compact(tab_hbm, hidx_hbm, out_hbm, idx_v, rows_v, out_v, sem):
    wid = _worker_id()
    base = wid * _CHUNK_A
    pltpu.sync_copy(hidx_hbm.at[pl.ds(2 * base, 2 * _CHUNK_A)], idx_v)
    pltpu.async_copy(tab_hbm.at[idx_v], rows_v, sem).wait()

    def body(i, carry):
        lanes = lax.iota(jnp.int32, 16) + i * 16
        f0 = plsc.load_gather(rows_v, [lanes * 2])
        f1 = plsc.load_gather(rows_v, [lanes * 2 + 1])
        out_v[pl.ds(i * 16, 16)] = _pack_bf16_pair(f0, f1)
        return carry

    lax.fori_loop(0, _CHUNK_A // 16, body, 0)
    pltpu.sync_copy(out_v, out_hbm.at[pl.ds(base, _CHUNK_A)])


_NSLICES = 4
_NSL = _N // _NSLICES
_NPW = _NSL // _NW
_CB = 2048
_NCH = _NPW // _CB


def _make_encode(slice_off):
    @functools.partial(
        pl.kernel,
        out_type=jax.ShapeDtypeStruct((_N_LEVELS, _NSL), jnp.int32),
        mesh=_MESH,
        scratch_types=[
            pltpu.VMEM((_GTOT,), jnp.int32),
            pltpu.VMEM((_CB,), jnp.float32),
            pltpu.VMEM((_CB,), jnp.float32),
            pltpu.VMEM((_CB,), jnp.float32),
            pltpu.VMEM((_CB,), jnp.float32),
            pltpu.VMEM((_CB,), jnp.float32),
            pltpu.VMEM((_CB,), jnp.float32),
            pltpu.VMEM((_CB * _N_LEVELS,), jnp.int32),
            pltpu.VMEM((_CB * _N_LEVELS,), jnp.int32),
            pltpu.SemaphoreType.DMA,
            pltpu.SemaphoreType.DMA,
            pltpu.SemaphoreType.DMA,
            pltpu.SemaphoreType.DMA,
        ],
        compiler_params=pltpu.CompilerParams(needs_layout_passes=False),
    )
    def _encode(x0_hbm, x1_hbm, x2_hbm, grid_hbm, pe_hbm, grid_v,
                xb00, xb01, xb02, xb10, xb11, xb12, pea, pebb,
                sx0, sx1, sp0, sp1):
        wid = _worker_id()
        base = wid * _NPW
        xbufs = [(xb00, xb01, xb02), (xb10, xb11, xb12)]
        xplanes = [x0_hbm, x1_hbm, x2_hbm]
        pebufs = [pea, pebb]
        sx = [sx0, sx1]
        sp = [sp0, sp1]

        def issue_x(ci, b):
            off = base + ci * _CB
            for c in range(3):
                pltpu.async_copy(xplanes[c].at[pl.ds(off, _CB)],
                                 xbufs[b][c], sx[b])

        def wait_x(b):
            for c in range(3):
                pltpu.make_async_copy(xplanes[c].at[pl.ds(0, _CB)],
                                      xbufs[b][c], sx[b]).wait()

        def issue_pe(ci, b):
            off = base + ci * _CB
            for l in range(_N_LEVELS):
                pltpu.async_copy(pebufs[b].at[pl.ds(l * _CB, _CB)],
                                 pe_hbm.at[l, pl.ds(off, _CB)], sp[b])

        def drain_pe(b):
            for l in range(_N_LEVELS):
                pltpu.make_async_copy(pebufs[b].at[pl.ds(l * _CB, _CB)],
                                      pe_hbm.at[l, pl.ds(base, _CB)], sp[b]).wait()

        issue_x(0, 0)
        pltpu.sync_copy(grid_hbm, grid_v)

        def chunk_pair(ci2, carry):
            for b in range(2):
                ci = ci2 * 2 + b
                peb = pebufs[b]
                xb0, xb1, xb2 = xbufs[b]
                wait_x(b)

                @pl.when(ci + 1 < _NCH)
                def _():
                    issue_x(ci + 1, 1 - b)

                @pl.when(ci >= 2)
                def _():
                    drain_pe(b)

                def g_body(g, gcarry):
                    p = g * 16
                    x0 = xb0[pl.ds(p, 16)]
                    x1 = xb1[pl.ds(p, 16)]
                    x2 = xb2[pl.ds(p, 16)]
                    for l in range(_N_LEVELS):
                        res = _RES[l]
                        s = _SIDES[l]
                        rf = jnp.float32(res)
                        s0 = x0 * rf
                        s1 = x1 * rf
                        s2 = x2 * rf
                        i0 = s0.astype(jnp.int32)
                        i1 = s1.astype(jnp.int32)
                        i2 = s2.astype(jnp.int32)
                        f0 = s0 - i0.astype(jnp.float32)
                        f1 = s1 - i1.astype(jnp.float32)
                        f2 = s2 - i2.astype(jnp.float32)
                        t0 = f0 * f0 * (3.0 - 2.0 * f0)
                        t1 = f1 * f1 * (3.0 - 2.0 * f1)
                        t2 = f2 * f2 * (3.0 - 2.0 * f2)
                        flat = (i0 * s + i1) * s + i2 + _LEVEL_OFFS[l]

                        tw0 = plsc.pack(t0, t0, format=plsc.PackFormat.INTERLEAVED)
                        tw1 = plsc.pack(t1, t1, format=plsc.PackFormat.INTERLEAVED)
                        tw2 = plsc.pack(t2, t2, format=plsc.PackFormat.INTERLEAVED)

                        def corner(d):
                            w = plsc.load_gather(grid_v, [flat + d])
                            return plsc.bitcast(w, jnp.bfloat16)

                        c000 = corner(0)
                        c001 = corner(1)
                        c010 = corner(s)
                        c011 = corner(s + 1)
                        c100 = corner(s * s)
                        c101 = corner(s * s + 1)
                        c110 = corner(s * s + s)
                        c111 = corner(s * s + s + 1)

                        def lerp(u, v, t):
                            return u + t * (v - u)

                        m00 = lerp(c000, c001, tw2)
                        m01 = lerp(c010, c011, tw2)
                        m10 = lerp(c100, c101, tw2)
                        m11 = lerp(c110, c111, tw2)
                        n0 = lerp(m00, m01, tw1)
                        n1 = lerp(m10, m11, tw1)
                        acc = lerp(n0, n1, tw0)

                        peb[pl.ds(l * _CB + p, 16)] = plsc.bitcast(acc, jnp.int32)
                    return gcarry

                def g_body_pl(g):
                    g_body(g, 0)

                plsc.parallel_loop(0, _CB // 16, unroll=2)(g_body_pl)
                issue_pe(ci, b)
            return carry

        lax.fori_loop(0, _NCH // 2, chunk_pair, 0)
        drain_pe(0)
        drain_pe(1)

    return _encode


_ENCODE = _make_encode(0)


_BLK = 4096


def _contract0(a, b):
    return lax.dot_general(a, b, (((0,), (0,)), ((), ())),
                           preferred_element_type=jnp.float32)


def _mlp_body(pe_ref, e_ref, x_ref, w1lo_ref, w1hi_ref, w1b_ref, w2_ref, w3_ref,
              b1_ref, b2_ref, b3_ref, o_ref):
    w = pe_ref[...]
    lo = lax.bitcast_convert_type(lax.shift_left(w, 16), jnp.float32)
    hi = lax.bitcast_convert_type(w & jnp.int32(-65536), jnp.float32)
    eb = e_ref[...].astype(jnp.bfloat16)
    h = _contract0(w1lo_ref[...], lo.astype(jnp.bfloat16))
    h = h + _contract0(w1hi_ref[...], hi.astype(jnp.bfloat16))
    h = h + _contract0(w1b_ref[...], eb)
    h = jnp.tanh(h + b1_ref[...])
    h2 = _contract0(w2_ref[...], h.astype(jnp.bfloat16))
    h2 = jnp.tanh(h2 + b2_ref[...])
    o = _contract0(w3_ref[...], h2.astype(jnp.bfloat16))
    o = o + b3_ref[...]
    xn = (x_ref[...] - _BB0) * (1.0 / (_BB1 - _BB0))
    o_ref[...] = (o + xn) * (_BB1 - _BB0) + _BB0


def _mlp(pe_t, e_t, x_t, W1lo, W1hi, W1b, W2b, W3b, b1r, b2r, b3r, blk_off):
    grid = (_NSL // _BLK,)
    return pl.pallas_call(
        _mlp_body,
        grid=grid,
        in_specs=[
            pl.BlockSpec((_N_LEVELS, _BLK), lambda i: (0, i)),
            pl.BlockSpec((_N_FEAT_E, _BLK), lambda i: (0, i + blk_off)),
            pl.BlockSpec((3, _BLK), lambda i: (0, i + blk_off)),
            pl.BlockSpec((_N_LEVELS, _WIDTH), lambda i: (0, 0)),
            pl.BlockSpec((_N_LEVELS, _WIDTH), lambda i: (0, 0)),
            pl.BlockSpec((_N_FEAT_E, _WIDTH), lambda i: (0, 0)),
            pl.BlockSpec((_WIDTH, _WIDTH), lambda i: (0, 0)),
            pl.BlockSpec((_WIDTH, 3), lambda i: (0, 0)),
            pl.BlockSpec((_WIDTH, 1), lambda i: (0, 0)),
            pl.BlockSpec((_WIDTH, 1), lambda i: (0, 0)),
            pl.BlockSpec((3, 1), lambda i: (0, 0)),
        ],
        out_specs=pl.BlockSpec((3, _BLK), lambda i: (0, i)),
        out_shape=jax.ShapeDtypeStruct((3, _NSL), jnp.float32),
    )(pe_t, e_t, x_t, W1lo, W1hi, W1b, W2b, W3b, b1r, b2r, b3r)


def kernel(x, e, tables, W1, b1, W2, b2, W3, b3):
    x_t = x.T
    e_t = e.T
    tab_flat = jnp.transpose(tables, (0, 2, 1)).reshape(_N_LEVELS * 2 * _T)
    hidx = jnp.asarray(_HIDX2)

    grid_packed = _compact(tab_flat, hidx)

    W1lo = W1[0:2 * _N_LEVELS:2].astype(jnp.bfloat16)
    W1hi = W1[1:2 * _N_LEVELS:2].astype(jnp.bfloat16)
    W1b = W1[2 * _N_LEVELS:].astype(jnp.bfloat16)
    W2b = W2.astype(jnp.bfloat16)
    W3b = W3.astype(jnp.bfloat16)
    b1r = b1.reshape(_WIDTH, 1)
    b2r = b2.reshape(_WIDTH, 1)
    b3r = b3.reshape(3, 1)

    parts = []
    for s in range(_NSLICES):
        sl = slice(s * _NSL, (s + 1) * _NSL)
        pe_s = _ENCODE(x[sl, 0], x[sl, 1], x[sl, 2], grid_packed)
        parts.append(_mlp(pe_s, e_t, x_t, W1lo, W1hi, W1b, W2b, W3b,
                          b1r, b2r, b3r, s * (_NSL // _BLK)))
    out_t = jnp.concatenate(parts, axis=1)
    return out_t.T.reshape(x.shape)
